# baseline (speedup 1.0000x reference)
import functools

import jax
import jax.numpy as jnp
from jax import lax
from jax.experimental import pallas as pl
from jax.experimental.pallas import tpu as pltpu
from jax.experimental.pallas import tpu_sc as plsc

N = 100000
S = 1000
SP = 1024
D = 512
DH = 1024
NC, NS, NW = 2, 16, 32
CH = 80
NCHUNK = N // CH
RB = 1000
NB = N // RB
EPS = 1e-5


GCH = 40
NCH2 = N // GCH
TROWS = 2048

def _gather_body(T_hbm, idx2_hbm, Agi_hbm, Egi_hbm,
                 buf0, buf1, idxb0, idxb1,
                 semi0, semi1, semg0, semg1, semw0, semw1):
    c = lax.axis_index("c")
    s = lax.axis_index("s")
    wid = s * NC + c
    c0 = (wid * NCH2) // NW
    c1 = ((wid + 1) * NCH2) // NW
    nmine = c1 - c0

    def start_i(i, idxb, sem):
        pltpu.async_copy(idx2_hbm.at[pl.ds((c0 + i) * 2 * GCH, 2 * GCH)],
                         idxb, sem)

    def wait_i(idxb, sem):
        pltpu.make_async_copy(idx2_hbm.at[pl.ds(0, 2 * GCH)], idxb, sem).wait()

    def start_g(buf, idxb, sem):
        pltpu.async_copy(T_hbm.at[idxb], buf, sem)

    def wait_g(buf, sem):
        pltpu.make_async_copy(T_hbm.at[pl.ds(0, 2 * GCH)], buf, sem).wait()

    def write_out(i, buf, sem):
        wa = pltpu.async_copy(buf.at[pl.ds(0, GCH)],
                              Agi_hbm.at[pl.ds((c0 + i) * GCH, GCH)], sem)
        we = pltpu.async_copy(buf.at[pl.ds(GCH, GCH)],
                              Egi_hbm.at[pl.ds((c0 + i) * GCH, GCH)], sem)
        return wa, we

    start_i(0, idxb0, semi0)
    wait_i(idxb0, semi0)
    start_g(buf0, idxb0, semg0)
    start_i(1, idxb1, semi1)

    def body(k, carry):
        a = 2 * k
        b = a + 1
        wait_g(buf0, semg0)

        @pl.when(b < nmine)
        def _():
            wait_i(idxb1, semi1)
            start_g(buf1, idxb1, semg1)

        wa, we = write_out(a, buf0, semw0)

        @pl.when(a + 2 < nmine)
        def _():
            start_i(a + 2, idxb0, semi0)

        @pl.when(b < nmine)
        def _():
            wait_g(buf1, semg1)

        wa.wait()
        we.wait()

        @pl.when(a + 2 < nmine)
        def _():
            wait_i(idxb0, semi0)
            start_g(buf0, idxb0, semg0)

        @pl.when(b < nmine)
        def _():
            wb, wee = write_out(b, buf1, semw1)

            @pl.when(b + 2 < nmine)
            def _():
                start_i(b + 2, idxb1, semi1)

            wb.wait()
            wee.wait()

        return carry

    lax.fori_loop(0, (nmine + 1) // 2, body, 0)


def _sc_gather(A, E, aidx, eidx):
    Ai = lax.bitcast_convert_type(A.reshape(S, DH // 2, 2), jnp.int32)
    Ei = lax.bitcast_convert_type(E.reshape(S, DH // 2, 2), jnp.int32)
    T = jnp.zeros((TROWS, DH // 2), jnp.int32)
    T = T.at[0:S].set(Ai).at[1024:1024 + S].set(Ei)
    a2 = aidx.reshape(NCH2, GCH)
    e2 = (eidx + 1024).reshape(NCH2, GCH)
    idx2 = jnp.concatenate([a2, e2], axis=1).reshape(-1)
    mesh = plsc.VectorSubcoreMesh(core_axis_name="c", subcore_axis_name="s")
    f = pl.kernel(
        _gather_body,
        out_type=(jax.ShapeDtypeStruct((N, DH // 2), jnp.int32),
                  jax.ShapeDtypeStruct((N, DH // 2), jnp.int32)),
        mesh=mesh,
        scratch_types=[
            pltpu.VMEM((2 * GCH, DH // 2), jnp.int32),
            pltpu.VMEM((2 * GCH, DH // 2), jnp.int32),
            pltpu.VMEM((2 * GCH,), jnp.int32),
            pltpu.VMEM((2 * GCH,), jnp.int32),
            pltpu.SemaphoreType.DMA,
            pltpu.SemaphoreType.DMA,
            pltpu.SemaphoreType.DMA,
            pltpu.SemaphoreType.DMA,
            pltpu.SemaphoreType.DMA,
            pltpu.SemaphoreType.DMA,
        ],
    )
    Agi, Egi = f(T, idx2)
    Ag = lax.bitcast_convert_type(Agi, jnp.bfloat16).reshape(N, DH)
    Eg = lax.bitcast_convert_type(Egi, jnp.bfloat16).reshape(N, DH)
    return Ag, Eg


def _tables_body(ps_ref, pc_ref, Wa_ref, We_ref, bA_ref, A_ref, E_ref):
    sums = ps_ref[0:S, :]
    cnt = pc_ref[0:S, 0:1].astype(jnp.float32)
    pooled = sums / jnp.maximum(cnt, 1.0)
    pa = pooled[:, 0:256]
    pe = pooled[:, 256:512]
    A_ref[...] = (jnp.dot(pa, Wa_ref[...], preferred_element_type=jnp.float32)
                  + bA_ref[...]).astype(jnp.bfloat16)
    E_ref[...] = jnp.dot(pe, We_ref[...],
                         preferred_element_type=jnp.float32).astype(jnp.bfloat16)


def _tc_tables(psums, pcnt, Wa, We, bA):
    return pl.pallas_call(
        _tables_body,
        out_shape=(jax.ShapeDtypeStruct((S, DH), jnp.bfloat16),
                   jax.ShapeDtypeStruct((S, DH), jnp.bfloat16)),
    )(psums, pcnt, Wa, We, bA)


def _h0pool_body(x_ref, rdf_ref, bdf_ref, idx_ref, Wx_ref, Wdr_ref, Wdb_ref,
                 H0_ref, ps_ref, pc_ref):
    xb = x_ref[...]
    o = jnp.dot(xb, Wx_ref[...], preferred_element_type=jnp.float32)
    dr = jnp.dot(rdf_ref[...], Wdr_ref[...], preferred_element_type=jnp.float32)
    db = jnp.dot(bdf_ref[...], Wdb_ref[...], preferred_element_type=jnp.float32)
    H0_ref[...] = o + jnp.concatenate([dr, db], axis=1)

    ids = idx_ref[0]
    ohT = (lax.broadcasted_iota(jnp.int32, (SP, RB), 0)
           == jnp.broadcast_to(ids, (SP, RB))).astype(jnp.bfloat16)
    ps = jnp.dot(ohT, xb.astype(jnp.bfloat16),
                 preferred_element_type=jnp.float32)
    pc = jnp.dot(ohT, jnp.ones((RB, 8), jnp.bfloat16),
                 preferred_element_type=jnp.float32)

    @pl.when(pl.program_id(0) == 0)
    def _():
        ps_ref[...] = ps
        pc_ref[...] = pc

    @pl.when(pl.program_id(0) != 0)
    def _():
        ps_ref[...] = ps_ref[...] + ps
        pc_ref[...] = pc_ref[...] + pc


def _tc_h0pool(x, rdf, bdf, aidx3, Wx, Wdr, Wdb):
    return pl.pallas_call(
        _h0pool_body,
        grid=(NB,),
        in_specs=[
            pl.BlockSpec((RB, D), lambda i: (i, 0)),
            pl.BlockSpec((RB, 128), lambda i: (i, 0)),
            pl.BlockSpec((RB, 128), lambda i: (i, 0)),
            pl.BlockSpec((1, 1, RB), lambda i: (i, 0, 0)),
            pl.BlockSpec((D, DH), lambda i: (0, 0)),
            pl.BlockSpec((128, D), lambda i: (0, 0)),
            pl.BlockSpec((128, D), lambda i: (0, 0)),
        ],
        out_specs=(pl.BlockSpec((RB, DH), lambda i: (i, 0)),
                   pl.BlockSpec((SP, D), lambda i: (0, 0)),
                   pl.BlockSpec((SP, 8), lambda i: (0, 0))),
        out_shape=(jax.ShapeDtypeStruct((N, DH), jnp.float32),
                   jax.ShapeDtypeStruct((SP, D), jnp.float32),
                   jax.ShapeDtypeStruct((SP, 8), jnp.float32)),
    )(x, rdf, bdf, aidx3, Wx, Wdr, Wdb)


def _stats1_body(H0_ref, Ag_ref, Eg_ref, st_ref):
    h = (H0_ref[...] + Ag_ref[...].astype(jnp.float32)
         + Eg_ref[...].astype(jnp.float32))
    ssum = jnp.sum(h, axis=0, keepdims=True)
    sqsum = jnp.sum(h * h, axis=0, keepdims=True)
    blk = jnp.concatenate([ssum, sqsum], axis=0)

    @pl.when(pl.program_id(0) == 0)
    def _():
        st_ref[...] = blk

    @pl.when(pl.program_id(0) != 0)
    def _():
        st_ref[...] = st_ref[...] + blk


def _tc_stats1(H0, Ag, Eg):
    return pl.pallas_call(
        _stats1_body,
        grid=(NB,),
        in_specs=[
            pl.BlockSpec((RB, DH), lambda i: (i, 0)),
            pl.BlockSpec((RB, DH), lambda i: (i, 0)),
            pl.BlockSpec((RB, DH), lambda i: (i, 0)),
        ],
        out_specs=pl.BlockSpec((2, DH), lambda i: (0, 0)),
        out_shape=jax.ShapeDtypeStruct((2, DH), jnp.float32),
    )(H0, Ag, Eg)


def _layer2_body(H0_ref, Ag_ref, Eg_ref, st_ref, g1_ref, bt1_ref,
                 W2_ref, b2_ref, h2_ref, st2_ref):
    nf = jnp.float32(N)
    mu = st_ref[0:1, :] / nf
    var = st_ref[1:2, :] / nf - mu * mu
    rstd = lax.rsqrt(var + EPS)
    scale = g1_ref[...] * rstd
    shift = bt1_ref[...] - mu * scale
    h1 = (H0_ref[...] + Ag_ref[...].astype(jnp.float32)
          + Eg_ref[...].astype(jnp.float32))
    x12 = jnp.maximum(h1 * scale + shift, 0.0)
    h2 = jnp.dot(x12, W2_ref[...],
                 preferred_element_type=jnp.float32) + b2_ref[...]
    h2_ref[...] = h2
    ssum = jnp.sum(h2, axis=0, keepdims=True)
    sqsum = jnp.sum(h2 * h2, axis=0, keepdims=True)
    blk = jnp.concatenate([ssum, sqsum], axis=0)

    @pl.when(pl.program_id(0) == 0)
    def _():
        st2_ref[...] = blk

    @pl.when(pl.program_id(0) != 0)
    def _():
        st2_ref[...] = st2_ref[...] + blk


def _tc_layer2(H0, Ag, Eg, st1, g1, bt1, W2, b2):
    return pl.pallas_call(
        _layer2_body,
        grid=(NB,),
        in_specs=[
            pl.BlockSpec((RB, DH), lambda i: (i, 0)),
            pl.BlockSpec((RB, DH), lambda i: (i, 0)),
            pl.BlockSpec((RB, DH), lambda i: (i, 0)),
            pl.BlockSpec((2, DH), lambda i: (0, 0)),
            pl.BlockSpec((1, DH), lambda i: (0, 0)),
            pl.BlockSpec((1, DH), lambda i: (0, 0)),
            pl.BlockSpec((DH, D), lambda i: (0, 0)),
            pl.BlockSpec((1, D), lambda i: (0, 0)),
        ],
        out_specs=(pl.BlockSpec((RB, D), lambda i: (i, 0)),
                   pl.BlockSpec((2, D), lambda i: (0, 0))),
        out_shape=(jax.ShapeDtypeStruct((N, D), jnp.float32),
                   jax.ShapeDtypeStruct((2, D), jnp.float32)),
    )(H0, Ag, Eg, st1, g1, bt1, W2, b2)


def _norm2_body(h2_ref, st2_ref, g2_ref, bt2_ref, out_ref):
    nf = jnp.float32(N)
    mu = st2_ref[0:1, :] / nf
    var = st2_ref[1:2, :] / nf - mu * mu
    rstd = lax.rsqrt(var + EPS)
    scale = g2_ref[...] * rstd
    shift = bt2_ref[...] - mu * scale
    out_ref[...] = jnp.maximum(h2_ref[...] * scale + shift, 0.0)


def _tc_norm2(h2, st2, g2, bt2):
    return pl.pallas_call(
        _norm2_body,
        grid=(NB,),
        in_specs=[
            pl.BlockSpec((RB, D), lambda i: (i, 0)),
            pl.BlockSpec((2, D), lambda i: (0, 0)),
            pl.BlockSpec((1, D), lambda i: (0, 0)),
            pl.BlockSpec((1, D), lambda i: (0, 0)),
        ],
        out_specs=pl.BlockSpec((RB, D), lambda i: (i, 0)),
        out_shape=jax.ShapeDtypeStruct((N, D), jnp.float32),
    )(h2, st2, g2, bt2)


def kernel(x, rdf_feat, bdf_feat, atom_idx, ele_idx,
           W1r, b1r, g1r, bt1r,
           W1b, b1b, g1b, bt1b,
           W2, b2, g2, bt2):
    aidx = atom_idx.astype(jnp.int32)
    eidx = ele_idx.astype(jnp.int32)

    Wx = jnp.concatenate(
        [jnp.concatenate([W1r[0:256], W1r[512:768]], axis=0),
         jnp.concatenate([W1b[0:256], W1b[512:768]], axis=0)], axis=1)
    Wdr = W1r[1024:1152]
    Wdb = W1b[1024:1152]
    Wa = jnp.concatenate([W1r[256:512], W1b[256:512]], axis=1)
    We = jnp.concatenate([W1r[768:1024], W1b[768:1024]], axis=1)
    bA = jnp.concatenate([b1r, b1b])[None, :]
    g1 = jnp.concatenate([g1r, g1b])[None, :]
    bt1 = jnp.concatenate([bt1r, bt1b])[None, :]

    aidx3 = aidx.reshape(NB, 1, RB)
    H0, psums, pcnt = _tc_h0pool(x, rdf_feat, bdf_feat, aidx3, Wx, Wdr, Wdb)
    A, E = _tc_tables(psums, pcnt, Wa, We, bA)
    Ag, Eg = _sc_gather(A, E, aidx, eidx)
    st1 = _tc_stats1(H0, Ag, Eg)
    h2, st2 = _tc_layer2(H0, Ag, Eg, st1, g1, bt1, W2, b2[None, :])
    return _tc_norm2(h2, st2, g2[None, :], bt2[None, :])


# speedup vs baseline: 2.8447x; 2.8447x over previous
import functools

import jax
import jax.numpy as jnp
from jax import lax
from jax.experimental import pallas as pl
from jax.experimental.pallas import tpu as pltpu
from jax.experimental.pallas import tpu_sc as plsc

N = 100000
S = 1000
SP = 1024
D = 512
DH = 1024
NC, NS, NW = 2, 16, 32
CH = 80
NCHUNK = N // CH
RB = 1000
NB = N // RB
EPS = 1e-5


GCH = 40
NCH2 = N // GCH
TROWS = 2048

def _gather_body(T_hbm, idx2_hbm, Agi_hbm, Egi_hbm,
                 buf0, buf1, idxb0, idxb1,
                 semi0, semi1, semg0, semg1, semw0, semw1):
    c = lax.axis_index("c")
    s = lax.axis_index("s")
    wid = s * NC + c
    c0 = (wid * NCH2) // NW
    c1 = ((wid + 1) * NCH2) // NW
    nmine = c1 - c0

    def start_i(i, idxb, sem):
        pltpu.async_copy(idx2_hbm.at[pl.ds((c0 + i) * 2 * GCH, 2 * GCH)],
                         idxb, sem)

    def wait_i(idxb, sem):
        pltpu.make_async_copy(idx2_hbm.at[pl.ds(0, 2 * GCH)], idxb, sem).wait()

    def start_g(buf, idxb, sem):
        pltpu.async_copy(T_hbm.at[idxb], buf, sem)

    def wait_g(buf, sem):
        pltpu.make_async_copy(T_hbm.at[pl.ds(0, 2 * GCH)], buf, sem).wait()

    def write_out(i, buf, sem):
        wa = pltpu.async_copy(buf.at[pl.ds(0, GCH)],
                              Agi_hbm.at[pl.ds((c0 + i) * GCH, GCH)], sem)
        we = pltpu.async_copy(buf.at[pl.ds(GCH, GCH)],
                              Egi_hbm.at[pl.ds((c0 + i) * GCH, GCH)], sem)
        return wa, we

    start_i(0, idxb0, semi0)
    wait_i(idxb0, semi0)
    start_g(buf0, idxb0, semg0)
    start_i(1, idxb1, semi1)

    def body(k, carry):
        a = 2 * k
        b = a + 1
        wait_g(buf0, semg0)

        @pl.when(b < nmine)
        def _():
            wait_i(idxb1, semi1)
            start_g(buf1, idxb1, semg1)

        wa, we = write_out(a, buf0, semw0)

        @pl.when(a + 2 < nmine)
        def _():
            start_i(a + 2, idxb0, semi0)

        @pl.when(b < nmine)
        def _():
            wait_g(buf1, semg1)

        wa.wait()
        we.wait()

        @pl.when(a + 2 < nmine)
        def _():
            wait_i(idxb0, semi0)
            start_g(buf0, idxb0, semg0)

        @pl.when(b < nmine)
        def _():
            wb, wee = write_out(b, buf1, semw1)

            @pl.when(b + 2 < nmine)
            def _():
                start_i(b + 2, idxb1, semi1)

            wb.wait()
            wee.wait()

        return carry

    lax.fori_loop(0, (nmine + 1) // 2, body, 0)


def _sc_gather(T, aidx, eidx):
    a2 = aidx.reshape(NCH2, GCH)
    e2 = (eidx + 1024).reshape(NCH2, GCH)
    idx2 = jnp.concatenate([a2, e2], axis=1).reshape(-1)
    mesh = plsc.VectorSubcoreMesh(core_axis_name="c", subcore_axis_name="s")
    f = pl.kernel(
        _gather_body,
        out_type=(jax.ShapeDtypeStruct((N, DH // 2), jnp.int32),
                  jax.ShapeDtypeStruct((N, DH // 2), jnp.int32)),
        mesh=mesh,
        scratch_types=[
            pltpu.VMEM((2 * GCH, DH // 2), jnp.int32),
            pltpu.VMEM((2 * GCH, DH // 2), jnp.int32),
            pltpu.VMEM((2 * GCH,), jnp.int32),
            pltpu.VMEM((2 * GCH,), jnp.int32),
            pltpu.SemaphoreType.DMA,
            pltpu.SemaphoreType.DMA,
            pltpu.SemaphoreType.DMA,
            pltpu.SemaphoreType.DMA,
            pltpu.SemaphoreType.DMA,
            pltpu.SemaphoreType.DMA,
        ],
    )
    return f(T, idx2)


def _pack16(v_lo, v_hi):
    blo = lax.bitcast_convert_type(
        v_lo.astype(jnp.bfloat16).astype(jnp.float32), jnp.uint32) >> 16
    bhi = lax.bitcast_convert_type(
        v_hi.astype(jnp.bfloat16).astype(jnp.float32), jnp.uint32) >> 16
    return lax.bitcast_convert_type((bhi << 16) | blo, jnp.int32)


def _unpack16(x):
    u = lax.bitcast_convert_type(x, jnp.uint32)
    lo = lax.bitcast_convert_type(u << 16, jnp.float32)
    hi = lax.bitcast_convert_type(u & jnp.uint32(0xFFFF0000), jnp.float32)
    return lo, hi


def _tables_body(ps_ref, pc_ref, Wa_ref, We_ref, bA_ref, T_ref):
    sums = ps_ref[0:S, :]
    cnt = pc_ref[0:S, 0:1].astype(jnp.float32)
    pooled = sums / jnp.maximum(cnt, 1.0)
    pa = pooled[:, 0:256]
    pe = pooled[:, 256:512]
    A = jnp.dot(pa, Wa_ref[...], preferred_element_type=jnp.float32) + bA_ref[...]
    E = jnp.dot(pe, We_ref[...], preferred_element_type=jnp.float32)
    T_ref[0:S, :] = _pack16(A[:, 0:512], A[:, 512:1024])
    T_ref[1024:1024 + S, :] = _pack16(E[:, 0:512], E[:, 512:1024])


def _tc_tables(psums, pcnt, Wa, We, bA):
    return pl.pallas_call(
        _tables_body,
        out_shape=jax.ShapeDtypeStruct((TROWS, DH // 2), jnp.int32),
    )(psums, pcnt, Wa, We, bA)


def _h0pool_body(x_ref, rdf_ref, bdf_ref, idx_ref, Wx_ref, Wdr_ref, Wdb_ref,
                 H0_ref, ps_ref, pc_ref):
    xb = x_ref[...]
    o = jnp.dot(xb, Wx_ref[...], preferred_element_type=jnp.float32)
    dr = jnp.dot(rdf_ref[...], Wdr_ref[...], preferred_element_type=jnp.float32)
    db = jnp.dot(bdf_ref[...], Wdb_ref[...], preferred_element_type=jnp.float32)
    H0_ref[...] = o + jnp.concatenate([dr, db], axis=1)

    ids = idx_ref[0]
    ohT = (lax.broadcasted_iota(jnp.int32, (SP, RB), 0)
           == jnp.broadcast_to(ids, (SP, RB))).astype(jnp.bfloat16)
    ps = jnp.dot(ohT, xb.astype(jnp.bfloat16),
                 preferred_element_type=jnp.float32)
    pc = jnp.dot(ohT, jnp.ones((RB, 8), jnp.bfloat16),
                 preferred_element_type=jnp.float32)

    @pl.when(pl.program_id(0) == 0)
    def _():
        ps_ref[...] = ps
        pc_ref[...] = pc

    @pl.when(pl.program_id(0) != 0)
    def _():
        ps_ref[...] = ps_ref[...] + ps
        pc_ref[...] = pc_ref[...] + pc


def _tc_h0pool(x, rdf, bdf, aidx3, Wx, Wdr, Wdb):
    return pl.pallas_call(
        _h0pool_body,
        grid=(NB,),
        in_specs=[
            pl.BlockSpec((RB, D), lambda i: (i, 0)),
            pl.BlockSpec((RB, 128), lambda i: (i, 0)),
            pl.BlockSpec((RB, 128), lambda i: (i, 0)),
            pl.BlockSpec((1, 1, RB), lambda i: (i, 0, 0)),
            pl.BlockSpec((D, DH), lambda i: (0, 0)),
            pl.BlockSpec((128, D), lambda i: (0, 0)),
            pl.BlockSpec((128, D), lambda i: (0, 0)),
        ],
        out_specs=(pl.BlockSpec((RB, DH), lambda i: (i, 0)),
                   pl.BlockSpec((SP, D), lambda i: (0, 0)),
                   pl.BlockSpec((SP, 8), lambda i: (0, 0))),
        out_shape=(jax.ShapeDtypeStruct((N, DH), jnp.float32),
                   jax.ShapeDtypeStruct((SP, D), jnp.float32),
                   jax.ShapeDtypeStruct((SP, 8), jnp.float32)),
    )(x, rdf, bdf, aidx3, Wx, Wdr, Wdb)


def _stats1_body(H0_ref, Ag_ref, Eg_ref, st_ref):
    alo, ahi = _unpack16(Ag_ref[...])
    elo, ehi = _unpack16(Eg_ref[...])
    ae = jnp.concatenate([alo + elo, ahi + ehi], axis=1)
    h = H0_ref[...] + ae
    ssum = jnp.sum(h, axis=0, keepdims=True)
    sqsum = jnp.sum(h * h, axis=0, keepdims=True)
    blk = jnp.concatenate([ssum, sqsum], axis=0)

    @pl.when(pl.program_id(0) == 0)
    def _():
        st_ref[...] = blk

    @pl.when(pl.program_id(0) != 0)
    def _():
        st_ref[...] = st_ref[...] + blk


def _tc_stats1(H0, Ag, Eg):
    return pl.pallas_call(
        _stats1_body,
        grid=(NB,),
        in_specs=[
            pl.BlockSpec((RB, DH), lambda i: (i, 0)),
            pl.BlockSpec((RB, DH // 2), lambda i: (i, 0)),
            pl.BlockSpec((RB, DH // 2), lambda i: (i, 0)),
        ],
        out_specs=pl.BlockSpec((2, DH), lambda i: (0, 0)),
        out_shape=jax.ShapeDtypeStruct((2, DH), jnp.float32),
    )(H0, Ag, Eg)


def _layer2_body(H0_ref, Ag_ref, Eg_ref, st_ref, g1_ref, bt1_ref,
                 W2_ref, b2_ref, h2_ref, st2_ref):
    nf = jnp.float32(N)
    mu = st_ref[0:1, :] / nf
    var = st_ref[1:2, :] / nf - mu * mu
    rstd = lax.rsqrt(var + EPS)
    scale = g1_ref[...] * rstd
    shift = bt1_ref[...] - mu * scale
    alo, ahi = _unpack16(Ag_ref[...])
    elo, ehi = _unpack16(Eg_ref[...])
    h1 = H0_ref[...] + jnp.concatenate([alo + elo, ahi + ehi], axis=1)
    x12 = jnp.maximum(h1 * scale + shift, 0.0)
    h2 = jnp.dot(x12, W2_ref[...],
                 preferred_element_type=jnp.float32) + b2_ref[...]
    h2_ref[...] = h2
    ssum = jnp.sum(h2, axis=0, keepdims=True)
    sqsum = jnp.sum(h2 * h2, axis=0, keepdims=True)
    blk = jnp.concatenate([ssum, sqsum], axis=0)

    @pl.when(pl.program_id(0) == 0)
    def _():
        st2_ref[...] = blk

    @pl.when(pl.program_id(0) != 0)
    def _():
        st2_ref[...] = st2_ref[...] + blk


def _tc_layer2(H0, Ag, Eg, st1, g1, bt1, W2, b2):
    return pl.pallas_call(
        _layer2_body,
        grid=(NB,),
        in_specs=[
            pl.BlockSpec((RB, DH), lambda i: (i, 0)),
            pl.BlockSpec((RB, DH // 2), lambda i: (i, 0)),
            pl.BlockSpec((RB, DH // 2), lambda i: (i, 0)),
            pl.BlockSpec((2, DH), lambda i: (0, 0)),
            pl.BlockSpec((1, DH), lambda i: (0, 0)),
            pl.BlockSpec((1, DH), lambda i: (0, 0)),
            pl.BlockSpec((DH, D), lambda i: (0, 0)),
            pl.BlockSpec((1, D), lambda i: (0, 0)),
        ],
        out_specs=(pl.BlockSpec((RB, D), lambda i: (i, 0)),
                   pl.BlockSpec((2, D), lambda i: (0, 0))),
        out_shape=(jax.ShapeDtypeStruct((N, D), jnp.float32),
                   jax.ShapeDtypeStruct((2, D), jnp.float32)),
    )(H0, Ag, Eg, st1, g1, bt1, W2, b2)


def _norm2_body(h2_ref, st2_ref, g2_ref, bt2_ref, out_ref):
    nf = jnp.float32(N)
    mu = st2_ref[0:1, :] / nf
    var = st2_ref[1:2, :] / nf - mu * mu
    rstd = lax.rsqrt(var + EPS)
    scale = g2_ref[...] * rstd
    shift = bt2_ref[...] - mu * scale
    out_ref[...] = jnp.maximum(h2_ref[...] * scale + shift, 0.0)


def _tc_norm2(h2, st2, g2, bt2):
    return pl.pallas_call(
        _norm2_body,
        grid=(NB,),
        in_specs=[
            pl.BlockSpec((RB, D), lambda i: (i, 0)),
            pl.BlockSpec((2, D), lambda i: (0, 0)),
            pl.BlockSpec((1, D), lambda i: (0, 0)),
            pl.BlockSpec((1, D), lambda i: (0, 0)),
        ],
        out_specs=pl.BlockSpec((RB, D), lambda i: (i, 0)),
        out_shape=jax.ShapeDtypeStruct((N, D), jnp.float32),
    )(h2, st2, g2, bt2)


def kernel(x, rdf_feat, bdf_feat, atom_idx, ele_idx,
           W1r, b1r, g1r, bt1r,
           W1b, b1b, g1b, bt1b,
           W2, b2, g2, bt2):
    aidx = atom_idx.astype(jnp.int32)
    eidx = ele_idx.astype(jnp.int32)

    Wx = jnp.concatenate(
        [jnp.concatenate([W1r[0:256], W1r[512:768]], axis=0),
         jnp.concatenate([W1b[0:256], W1b[512:768]], axis=0)], axis=1)
    Wdr = W1r[1024:1152]
    Wdb = W1b[1024:1152]
    Wa = jnp.concatenate([W1r[256:512], W1b[256:512]], axis=1)
    We = jnp.concatenate([W1r[768:1024], W1b[768:1024]], axis=1)
    bA = jnp.concatenate([b1r, b1b])[None, :]
    g1 = jnp.concatenate([g1r, g1b])[None, :]
    bt1 = jnp.concatenate([bt1r, bt1b])[None, :]

    aidx3 = aidx.reshape(NB, 1, RB)
    H0, psums, pcnt = _tc_h0pool(x, rdf_feat, bdf_feat, aidx3, Wx, Wdr, Wdb)
    T = _tc_tables(psums, pcnt, Wa, We, bA)
    Ag, Eg = _sc_gather(T, aidx, eidx)
    st1 = _tc_stats1(H0, Ag, Eg)
    h2, st2 = _tc_layer2(H0, Ag, Eg, st1, g1, bt1, W2, b2[None, :])
    return _tc_norm2(h2, st2, g2[None, :], bt2[None, :])


# speedup vs baseline: 2.9620x; 1.0413x over previous
import functools

import jax
import jax.numpy as jnp
from jax import lax
from jax.experimental import pallas as pl
from jax.experimental.pallas import tpu as pltpu
from jax.experimental.pallas import tpu_sc as plsc

N = 100000
S = 1000
SP = 1024
D = 512
DH = 1024
NC, NS, NW = 2, 16, 32
CH = 80
NCHUNK = N // CH
RB = 1000
NB = N // RB
EPS = 1e-5


GCH = 40
NCH2 = N // GCH
TROWS = 2048

def _gather_body(T_hbm, idx2_hbm, Agi_hbm, Egi_hbm,
                 buf0, buf1, idxb0, idxb1,
                 semi0, semi1, semg0, semg1, semw0, semw1):
    c = lax.axis_index("c")
    s = lax.axis_index("s")
    wid = s * NC + c
    c0 = (wid * NCH2) // NW
    c1 = ((wid + 1) * NCH2) // NW
    nmine = c1 - c0

    def start_i(i, idxb, sem):
        pltpu.async_copy(idx2_hbm.at[pl.ds((c0 + i) * 2 * GCH, 2 * GCH)],
                         idxb, sem)

    def wait_i(idxb, sem):
        pltpu.make_async_copy(idx2_hbm.at[pl.ds(0, 2 * GCH)], idxb, sem).wait()

    def start_g(buf, idxb, sem):
        pltpu.async_copy(T_hbm.at[idxb], buf, sem)

    def wait_g(buf, sem):
        pltpu.make_async_copy(T_hbm.at[pl.ds(0, 2 * GCH)], buf, sem).wait()

    def write_out(i, buf, sem):
        wa = pltpu.async_copy(buf.at[pl.ds(0, GCH)],
                              Agi_hbm.at[pl.ds((c0 + i) * GCH, GCH)], sem)
        we = pltpu.async_copy(buf.at[pl.ds(GCH, GCH)],
                              Egi_hbm.at[pl.ds((c0 + i) * GCH, GCH)], sem)
        return wa, we

    start_i(0, idxb0, semi0)
    wait_i(idxb0, semi0)
    start_g(buf0, idxb0, semg0)
    start_i(1, idxb1, semi1)

    def body(k, carry):
        a = 2 * k
        b = a + 1
        wait_g(buf0, semg0)

        @pl.when(b < nmine)
        def _():
            wait_i(idxb1, semi1)
            start_g(buf1, idxb1, semg1)

        wa, we = write_out(a, buf0, semw0)

        @pl.when(a + 2 < nmine)
        def _():
            start_i(a + 2, idxb0, semi0)

        @pl.when(b < nmine)
        def _():
            wait_g(buf1, semg1)

        wa.wait()
        we.wait()

        @pl.when(a + 2 < nmine)
        def _():
            wait_i(idxb0, semi0)
            start_g(buf0, idxb0, semg0)

        @pl.when(b < nmine)
        def _():
            wb, wee = write_out(b, buf1, semw1)

            @pl.when(b + 2 < nmine)
            def _():
                start_i(b + 2, idxb1, semi1)

            wb.wait()
            wee.wait()

        return carry

    lax.fori_loop(0, (nmine + 1) // 2, body, 0)


def _sc_gather(T, aidx, eidx):
    a2 = aidx.reshape(NCH2, GCH)
    e2 = (eidx + 1024).reshape(NCH2, GCH)
    idx2 = jnp.concatenate([a2, e2], axis=1).reshape(-1)
    mesh = plsc.VectorSubcoreMesh(core_axis_name="c", subcore_axis_name="s")
    f = pl.kernel(
        _gather_body,
        out_type=(jax.ShapeDtypeStruct((N, DH // 2), jnp.int32),
                  jax.ShapeDtypeStruct((N, DH // 2), jnp.int32)),
        mesh=mesh,
        scratch_types=[
            pltpu.VMEM((2 * GCH, DH // 2), jnp.int32),
            pltpu.VMEM((2 * GCH, DH // 2), jnp.int32),
            pltpu.VMEM((2 * GCH,), jnp.int32),
            pltpu.VMEM((2 * GCH,), jnp.int32),
            pltpu.SemaphoreType.DMA,
            pltpu.SemaphoreType.DMA,
            pltpu.SemaphoreType.DMA,
            pltpu.SemaphoreType.DMA,
            pltpu.SemaphoreType.DMA,
            pltpu.SemaphoreType.DMA,
        ],
    )
    return f(T, idx2)


def _pack16(v_lo, v_hi):
    blo = lax.bitcast_convert_type(
        v_lo.astype(jnp.bfloat16).astype(jnp.float32), jnp.uint32) >> 16
    bhi = lax.bitcast_convert_type(
        v_hi.astype(jnp.bfloat16).astype(jnp.float32), jnp.uint32) >> 16
    return lax.bitcast_convert_type((bhi << 16) | blo, jnp.int32)


def _unpack16(x):
    u = lax.bitcast_convert_type(x, jnp.uint32)
    lo = lax.bitcast_convert_type(u << 16, jnp.float32)
    hi = lax.bitcast_convert_type(u & jnp.uint32(0xFFFF0000), jnp.float32)
    return lo, hi


def _tables_body(ps_ref, pc_ref, Wa_ref, We_ref, bA_ref, T_ref):
    sums = ps_ref[0:S, :]
    cnt = pc_ref[0:S, 0:1].astype(jnp.float32)
    pooled = sums / jnp.maximum(cnt, 1.0)
    pa = pooled[:, 0:256]
    pe = pooled[:, 256:512]
    A = jnp.dot(pa, Wa_ref[...], preferred_element_type=jnp.float32) + bA_ref[...]
    E = jnp.dot(pe, We_ref[...], preferred_element_type=jnp.float32)
    T_ref[0:S, :] = _pack16(A[:, 0:512], A[:, 512:1024])
    T_ref[1024:1024 + S, :] = _pack16(E[:, 0:512], E[:, 512:1024])


def _tc_tables(psums, pcnt, Wa, We, bA):
    return pl.pallas_call(
        _tables_body,
        out_shape=jax.ShapeDtypeStruct((TROWS, DH // 2), jnp.int32),
    )(psums, pcnt, Wa, We, bA)


def _h0pool_body(x_ref, rdf_ref, bdf_ref, idx_ref, Wx_ref, Wdr_ref, Wdb_ref,
                 H0_ref, ps_ref, pc_ref):
    xb = x_ref[...]
    xb16 = xb.astype(jnp.bfloat16)
    o = jnp.dot(xb16, Wx_ref[...].astype(jnp.bfloat16),
                preferred_element_type=jnp.float32)
    dr = jnp.dot(rdf_ref[...].astype(jnp.bfloat16),
                 Wdr_ref[...].astype(jnp.bfloat16),
                 preferred_element_type=jnp.float32)
    db = jnp.dot(bdf_ref[...].astype(jnp.bfloat16),
                 Wdb_ref[...].astype(jnp.bfloat16),
                 preferred_element_type=jnp.float32)
    H0_ref[...] = (o + jnp.concatenate([dr, db], axis=1)).astype(jnp.bfloat16)

    ids = idx_ref[0]
    ohT = (lax.broadcasted_iota(jnp.int32, (SP, RB), 0)
           == jnp.broadcast_to(ids, (SP, RB))).astype(jnp.bfloat16)
    ps = jnp.dot(ohT, xb16, preferred_element_type=jnp.float32)
    pc = jnp.dot(ohT, jnp.ones((RB, 8), jnp.bfloat16),
                 preferred_element_type=jnp.float32)

    @pl.when(pl.program_id(0) == 0)
    def _():
        ps_ref[...] = ps
        pc_ref[...] = pc

    @pl.when(pl.program_id(0) != 0)
    def _():
        ps_ref[...] = ps_ref[...] + ps
        pc_ref[...] = pc_ref[...] + pc


def _tc_h0pool(x, rdf, bdf, aidx3, Wx, Wdr, Wdb):
    return pl.pallas_call(
        _h0pool_body,
        grid=(NB,),
        in_specs=[
            pl.BlockSpec((RB, D), lambda i: (i, 0)),
            pl.BlockSpec((RB, 128), lambda i: (i, 0)),
            pl.BlockSpec((RB, 128), lambda i: (i, 0)),
            pl.BlockSpec((1, 1, RB), lambda i: (i, 0, 0)),
            pl.BlockSpec((D, DH), lambda i: (0, 0)),
            pl.BlockSpec((128, D), lambda i: (0, 0)),
            pl.BlockSpec((128, D), lambda i: (0, 0)),
        ],
        out_specs=(pl.BlockSpec((RB, DH), lambda i: (i, 0)),
                   pl.BlockSpec((SP, D), lambda i: (0, 0)),
                   pl.BlockSpec((SP, 8), lambda i: (0, 0))),
        out_shape=(jax.ShapeDtypeStruct((N, DH), jnp.bfloat16),
                   jax.ShapeDtypeStruct((SP, D), jnp.float32),
                   jax.ShapeDtypeStruct((SP, 8), jnp.float32)),
    )(x, rdf, bdf, aidx3, Wx, Wdr, Wdb)


def _stats1_body(H0_ref, Ag_ref, Eg_ref, st_ref):
    alo, ahi = _unpack16(Ag_ref[...])
    elo, ehi = _unpack16(Eg_ref[...])
    ae = jnp.concatenate([alo + elo, ahi + ehi], axis=1)
    h = H0_ref[...].astype(jnp.float32) + ae
    ssum = jnp.sum(h, axis=0, keepdims=True)
    sqsum = jnp.sum(h * h, axis=0, keepdims=True)
    blk = jnp.concatenate([ssum, sqsum], axis=0)

    @pl.when(pl.program_id(0) == 0)
    def _():
        st_ref[...] = blk

    @pl.when(pl.program_id(0) != 0)
    def _():
        st_ref[...] = st_ref[...] + blk


def _tc_stats1(H0, Ag, Eg):
    return pl.pallas_call(
        _stats1_body,
        grid=(NB,),
        in_specs=[
            pl.BlockSpec((RB, DH), lambda i: (i, 0)),
            pl.BlockSpec((RB, DH // 2), lambda i: (i, 0)),
            pl.BlockSpec((RB, DH // 2), lambda i: (i, 0)),
        ],
        out_specs=pl.BlockSpec((2, DH), lambda i: (0, 0)),
        out_shape=jax.ShapeDtypeStruct((2, DH), jnp.float32),
    )(H0, Ag, Eg)


def _layer2_body(H0_ref, Ag_ref, Eg_ref, st_ref, g1_ref, bt1_ref,
                 W2_ref, b2_ref, h2_ref, st2_ref):
    nf = jnp.float32(N)
    mu = st_ref[0:1, :] / nf
    var = st_ref[1:2, :] / nf - mu * mu
    rstd = lax.rsqrt(var + EPS)
    scale = g1_ref[...] * rstd
    shift = bt1_ref[...] - mu * scale
    alo, ahi = _unpack16(Ag_ref[...])
    elo, ehi = _unpack16(Eg_ref[...])
    h1 = (H0_ref[...].astype(jnp.float32)
          + jnp.concatenate([alo + elo, ahi + ehi], axis=1))
    x12 = jnp.maximum(h1 * scale + shift, 0.0)
    h2 = jnp.dot(x12.astype(jnp.bfloat16), W2_ref[...].astype(jnp.bfloat16),
                 preferred_element_type=jnp.float32) + b2_ref[...]
    h2_ref[...] = h2
    ssum = jnp.sum(h2, axis=0, keepdims=True)
    sqsum = jnp.sum(h2 * h2, axis=0, keepdims=True)
    blk = jnp.concatenate([ssum, sqsum], axis=0)

    @pl.when(pl.program_id(0) == 0)
    def _():
        st2_ref[...] = blk

    @pl.when(pl.program_id(0) != 0)
    def _():
        st2_ref[...] = st2_ref[...] + blk


def _tc_layer2(H0, Ag, Eg, st1, g1, bt1, W2, b2):
    return pl.pallas_call(
        _layer2_body,
        grid=(NB,),
        in_specs=[
            pl.BlockSpec((RB, DH), lambda i: (i, 0)),
            pl.BlockSpec((RB, DH // 2), lambda i: (i, 0)),
            pl.BlockSpec((RB, DH // 2), lambda i: (i, 0)),
            pl.BlockSpec((2, DH), lambda i: (0, 0)),
            pl.BlockSpec((1, DH), lambda i: (0, 0)),
            pl.BlockSpec((1, DH), lambda i: (0, 0)),
            pl.BlockSpec((DH, D), lambda i: (0, 0)),
            pl.BlockSpec((1, D), lambda i: (0, 0)),
        ],
        out_specs=(pl.BlockSpec((RB, D), lambda i: (i, 0)),
                   pl.BlockSpec((2, D), lambda i: (0, 0))),
        out_shape=(jax.ShapeDtypeStruct((N, D), jnp.float32),
                   jax.ShapeDtypeStruct((2, D), jnp.float32)),
    )(H0, Ag, Eg, st1, g1, bt1, W2, b2)


def _norm2_body(h2_ref, st2_ref, g2_ref, bt2_ref, out_ref):
    nf = jnp.float32(N)
    mu = st2_ref[0:1, :] / nf
    var = st2_ref[1:2, :] / nf - mu * mu
    rstd = lax.rsqrt(var + EPS)
    scale = g2_ref[...] * rstd
    shift = bt2_ref[...] - mu * scale
    out_ref[...] = jnp.maximum(h2_ref[...] * scale + shift, 0.0)


def _tc_norm2(h2, st2, g2, bt2):
    return pl.pallas_call(
        _norm2_body,
        grid=(NB,),
        in_specs=[
            pl.BlockSpec((RB, D), lambda i: (i, 0)),
            pl.BlockSpec((2, D), lambda i: (0, 0)),
            pl.BlockSpec((1, D), lambda i: (0, 0)),
            pl.BlockSpec((1, D), lambda i: (0, 0)),
        ],
        out_specs=pl.BlockSpec((RB, D), lambda i: (i, 0)),
        out_shape=jax.ShapeDtypeStruct((N, D), jnp.float32),
    )(h2, st2, g2, bt2)


def kernel(x, rdf_feat, bdf_feat, atom_idx, ele_idx,
           W1r, b1r, g1r, bt1r,
           W1b, b1b, g1b, bt1b,
           W2, b2, g2, bt2):
    aidx = atom_idx.astype(jnp.int32)
    eidx = ele_idx.astype(jnp.int32)

    Wx = jnp.concatenate(
        [jnp.concatenate([W1r[0:256], W1r[512:768]], axis=0),
         jnp.concatenate([W1b[0:256], W1b[512:768]], axis=0)], axis=1)
    Wdr = W1r[1024:1152]
    Wdb = W1b[1024:1152]
    Wa = jnp.concatenate([W1r[256:512], W1b[256:512]], axis=1)
    We = jnp.concatenate([W1r[768:1024], W1b[768:1024]], axis=1)
    bA = jnp.concatenate([b1r, b1b])[None, :]
    g1 = jnp.concatenate([g1r, g1b])[None, :]
    bt1 = jnp.concatenate([bt1r, bt1b])[None, :]

    aidx3 = aidx.reshape(NB, 1, RB)
    H0, psums, pcnt = _tc_h0pool(x, rdf_feat, bdf_feat, aidx3, Wx, Wdr, Wdb)
    T = _tc_tables(psums, pcnt, Wa, We, bA)
    Ag, Eg = _sc_gather(T, aidx, eidx)
    st1 = _tc_stats1(H0, Ag, Eg)
    h2, st2 = _tc_layer2(H0, Ag, Eg, st1, g1, bt1, W2, b2[None, :])
    return _tc_norm2(h2, st2, g2[None, :], bt2[None, :])


# speedup vs baseline: 4.0806x; 1.3776x over previous
import functools

import jax
import jax.numpy as jnp
from jax import lax
from jax.experimental import pallas as pl
from jax.experimental.pallas import tpu as pltpu
from jax.experimental.pallas import tpu_sc as plsc

N = 100000
S = 1000
SP = 1024
D = 512
DH = 1024
NC, NS, NW = 2, 16, 32
CH = 80
NCHUNK = N // CH
RB = 1000
NB = N // RB
EPS = 1e-5


GCH = 80
NCH2 = N // GCH
TROWS = 2048

def _gather_body(T_hbm, idx2_hbm, Egi_hbm,
                 buf0, buf1, idxb0, idxb1,
                 semi0, semi1, semg0, semg1, semw0, semw1):
    c = lax.axis_index("c")
    s = lax.axis_index("s")
    wid = s * NC + c
    c0 = (wid * NCH2) // NW
    c1 = ((wid + 1) * NCH2) // NW
    nmine = c1 - c0

    def start_i(i, idxb, sem):
        pltpu.async_copy(idx2_hbm.at[pl.ds((c0 + i) * GCH, GCH)], idxb, sem)

    def wait_i(idxb, sem):
        pltpu.make_async_copy(idx2_hbm.at[pl.ds(0, GCH)], idxb, sem).wait()

    def start_g(buf, idxb, sem):
        pltpu.async_copy(T_hbm.at[idxb], buf, sem)

    def wait_g(buf, sem):
        pltpu.make_async_copy(T_hbm.at[pl.ds(0, GCH)], buf, sem).wait()

    def write_out(i, buf, sem):
        we = pltpu.async_copy(buf, Egi_hbm.at[pl.ds((c0 + i) * GCH, GCH)], sem)
        return we

    start_i(0, idxb0, semi0)
    wait_i(idxb0, semi0)
    start_g(buf0, idxb0, semg0)
    start_i(1, idxb1, semi1)

    def body(k, carry):
        a = 2 * k
        b = a + 1
        wait_g(buf0, semg0)

        @pl.when(b < nmine)
        def _():
            wait_i(idxb1, semi1)
            start_g(buf1, idxb1, semg1)

        wa = write_out(a, buf0, semw0)

        @pl.when(a + 2 < nmine)
        def _():
            start_i(a + 2, idxb0, semi0)

        @pl.when(b < nmine)
        def _():
            wait_g(buf1, semg1)

        wa.wait()

        @pl.when(a + 2 < nmine)
        def _():
            wait_i(idxb0, semi0)
            start_g(buf0, idxb0, semg0)

        @pl.when(b < nmine)
        def _():
            wb = write_out(b, buf1, semw1)

            @pl.when(b + 2 < nmine)
            def _():
                start_i(b + 2, idxb1, semi1)

            wb.wait()

        return carry

    lax.fori_loop(0, (nmine + 1) // 2, body, 0)


def _sc_gather(T, eidx):
    idx2 = eidx + 1024
    mesh = plsc.VectorSubcoreMesh(core_axis_name="c", subcore_axis_name="s")
    f = pl.kernel(
        _gather_body,
        out_type=jax.ShapeDtypeStruct((N, DH // 2), jnp.int32),
        mesh=mesh,
        scratch_types=[
            pltpu.VMEM((GCH, DH // 2), jnp.int32),
            pltpu.VMEM((GCH, DH // 2), jnp.int32),
            pltpu.VMEM((GCH,), jnp.int32),
            pltpu.VMEM((GCH,), jnp.int32),
            pltpu.SemaphoreType.DMA,
            pltpu.SemaphoreType.DMA,
            pltpu.SemaphoreType.DMA,
            pltpu.SemaphoreType.DMA,
            pltpu.SemaphoreType.DMA,
            pltpu.SemaphoreType.DMA,
        ],
    )
    return f(T, idx2)


def _pack16(v_lo, v_hi):
    blo = lax.bitcast_convert_type(
        v_lo.astype(jnp.bfloat16).astype(jnp.float32), jnp.uint32) >> 16
    bhi = lax.bitcast_convert_type(
        v_hi.astype(jnp.bfloat16).astype(jnp.float32), jnp.uint32) >> 16
    return lax.bitcast_convert_type((bhi << 16) | blo, jnp.int32)


def _unpack16(x):
    u = lax.bitcast_convert_type(x, jnp.uint32)
    lo = lax.bitcast_convert_type(u << 16, jnp.float32)
    hi = lax.bitcast_convert_type(u & jnp.uint32(0xFFFF0000), jnp.float32)
    return lo, hi


def _tables_body(ps_ref, pc_ref, Wa_ref, We_ref, bA_ref, T_ref):
    sums = ps_ref[0:S, :]
    cnt = pc_ref[0:S, 0:1].astype(jnp.float32)
    pooled = sums / jnp.maximum(cnt, 1.0)
    pa = pooled[:, 0:256]
    pe = pooled[:, 256:512]
    A = jnp.dot(pa, Wa_ref[...], preferred_element_type=jnp.float32) + bA_ref[...]
    E = jnp.dot(pe, We_ref[...], preferred_element_type=jnp.float32)
    T_ref[0:S, :] = _pack16(A[:, 0:512], A[:, 512:1024])
    T_ref[1024:1024 + S, :] = _pack16(E[:, 0:512], E[:, 512:1024])


def _tc_tables(psums, pcnt, Wa, We, bA):
    return pl.pallas_call(
        _tables_body,
        out_shape=jax.ShapeDtypeStruct((TROWS, DH // 2), jnp.int32),
    )(psums, pcnt, Wa, We, bA)


def _pool_body(x_ref, idx_ref, ps_ref, pc_ref):
    xb16 = x_ref[...].astype(jnp.bfloat16)
    ids = idx_ref[0]
    ohT = (lax.broadcasted_iota(jnp.int32, (SP, RB), 0)
           == jnp.broadcast_to(ids, (SP, RB))).astype(jnp.bfloat16)
    ps = jnp.dot(ohT, xb16, preferred_element_type=jnp.float32)
    pc = jnp.dot(ohT, jnp.ones((RB, 8), jnp.bfloat16),
                 preferred_element_type=jnp.float32)

    @pl.when(pl.program_id(0) == 0)
    def _():
        ps_ref[...] = ps
        pc_ref[...] = pc

    @pl.when(pl.program_id(0) != 0)
    def _():
        ps_ref[...] = ps_ref[...] + ps
        pc_ref[...] = pc_ref[...] + pc


def _tc_pool(x, aidx3):
    return pl.pallas_call(
        _pool_body,
        grid=(NB,),
        in_specs=[
            pl.BlockSpec((RB, D), lambda i: (i, 0)),
            pl.BlockSpec((1, 1, RB), lambda i: (i, 0, 0)),
        ],
        out_specs=(pl.BlockSpec((SP, D), lambda i: (0, 0)),
                   pl.BlockSpec((SP, 8), lambda i: (0, 0))),
        out_shape=(jax.ShapeDtypeStruct((SP, D), jnp.float32),
                   jax.ShapeDtypeStruct((SP, 8), jnp.float32)),
    )(x, aidx3)


def _h0a_body(x_ref, rdf_ref, bdf_ref, idx_ref, Wx_ref, Wdr_ref, Wdb_ref,
              T_ref, H0_ref):
    xb16 = x_ref[...].astype(jnp.bfloat16)
    o = jnp.dot(xb16, Wx_ref[...].astype(jnp.bfloat16),
                preferred_element_type=jnp.float32)
    dr = jnp.dot(rdf_ref[...].astype(jnp.bfloat16),
                 Wdr_ref[...].astype(jnp.bfloat16),
                 preferred_element_type=jnp.float32)
    db = jnp.dot(bdf_ref[...].astype(jnp.bfloat16),
                 Wdb_ref[...].astype(jnp.bfloat16),
                 preferred_element_type=jnp.float32)
    alo, ahi = _unpack16(T_ref[...])
    Abf = jnp.concatenate([alo, ahi], axis=1).astype(jnp.bfloat16)
    ids = idx_ref[0]
    ohT = (lax.broadcasted_iota(jnp.int32, (SP, RB), 0)
           == jnp.broadcast_to(ids, (SP, RB))).astype(jnp.bfloat16)
    ag = lax.dot_general(ohT, Abf, (((0,), (0,)), ((), ())),
                         preferred_element_type=jnp.float32)
    H0_ref[...] = (o + ag
                   + jnp.concatenate([dr, db], axis=1)).astype(jnp.bfloat16)


def _tc_h0a(x, rdf, bdf, aidx3, Wx, Wdr, Wdb, T):
    return pl.pallas_call(
        _h0a_body,
        grid=(NB,),
        in_specs=[
            pl.BlockSpec((RB, D), lambda i: (i, 0)),
            pl.BlockSpec((RB, 128), lambda i: (i, 0)),
            pl.BlockSpec((RB, 128), lambda i: (i, 0)),
            pl.BlockSpec((1, 1, RB), lambda i: (i, 0, 0)),
            pl.BlockSpec((D, DH), lambda i: (0, 0)),
            pl.BlockSpec((128, D), lambda i: (0, 0)),
            pl.BlockSpec((128, D), lambda i: (0, 0)),
            pl.BlockSpec((SP, DH // 2), lambda i: (0, 0)),
        ],
        out_specs=pl.BlockSpec((RB, DH), lambda i: (i, 0)),
        out_shape=jax.ShapeDtypeStruct((N, DH), jnp.bfloat16),
    )(x, rdf, bdf, aidx3, Wx, Wdr, Wdb, T)


def _stats1_body(H0_ref, Eg_ref, st_ref):
    elo, ehi = _unpack16(Eg_ref[...])
    h = H0_ref[...].astype(jnp.float32) + jnp.concatenate([elo, ehi], axis=1)
    ssum = jnp.sum(h, axis=0, keepdims=True)
    sqsum = jnp.sum(h * h, axis=0, keepdims=True)
    blk = jnp.concatenate([ssum, sqsum], axis=0)

    @pl.when(pl.program_id(0) == 0)
    def _():
        st_ref[...] = blk

    @pl.when(pl.program_id(0) != 0)
    def _():
        st_ref[...] = st_ref[...] + blk


def _tc_stats1(H0, Eg):
    return pl.pallas_call(
        _stats1_body,
        grid=(NB,),
        in_specs=[
            pl.BlockSpec((RB, DH), lambda i: (i, 0)),
            pl.BlockSpec((RB, DH // 2), lambda i: (i, 0)),
        ],
        out_specs=pl.BlockSpec((2, DH), lambda i: (0, 0)),
        out_shape=jax.ShapeDtypeStruct((2, DH), jnp.float32),
    )(H0, Eg)


def _layer2_body(H0_ref, Eg_ref, st_ref, g1_ref, bt1_ref,
                 W2_ref, b2_ref, h2_ref, st2_ref):
    nf = jnp.float32(N)
    mu = st_ref[0:1, :] / nf
    var = st_ref[1:2, :] / nf - mu * mu
    rstd = lax.rsqrt(var + EPS)
    scale = g1_ref[...] * rstd
    shift = bt1_ref[...] - mu * scale
    elo, ehi = _unpack16(Eg_ref[...])
    h1 = (H0_ref[...].astype(jnp.float32)
          + jnp.concatenate([elo, ehi], axis=1))
    x12 = jnp.maximum(h1 * scale + shift, 0.0)
    h2 = jnp.dot(x12.astype(jnp.bfloat16), W2_ref[...].astype(jnp.bfloat16),
                 preferred_element_type=jnp.float32) + b2_ref[...]
    h2_ref[...] = h2
    ssum = jnp.sum(h2, axis=0, keepdims=True)
    sqsum = jnp.sum(h2 * h2, axis=0, keepdims=True)
    blk = jnp.concatenate([ssum, sqsum], axis=0)

    @pl.when(pl.program_id(0) == 0)
    def _():
        st2_ref[...] = blk

    @pl.when(pl.program_id(0) != 0)
    def _():
        st2_ref[...] = st2_ref[...] + blk


def _tc_layer2(H0, Eg, st1, g1, bt1, W2, b2):
    return pl.pallas_call(
        _layer2_body,
        grid=(NB,),
        in_specs=[
            pl.BlockSpec((RB, DH), lambda i: (i, 0)),
            pl.BlockSpec((RB, DH // 2), lambda i: (i, 0)),
            pl.BlockSpec((2, DH), lambda i: (0, 0)),
            pl.BlockSpec((1, DH), lambda i: (0, 0)),
            pl.BlockSpec((1, DH), lambda i: (0, 0)),
            pl.BlockSpec((DH, D), lambda i: (0, 0)),
            pl.BlockSpec((1, D), lambda i: (0, 0)),
        ],
        out_specs=(pl.BlockSpec((RB, D), lambda i: (i, 0)),
                   pl.BlockSpec((2, D), lambda i: (0, 0))),
        out_shape=(jax.ShapeDtypeStruct((N, D), jnp.float32),
                   jax.ShapeDtypeStruct((2, D), jnp.float32)),
    )(H0, Eg, st1, g1, bt1, W2, b2)


def _norm2_body(h2_ref, st2_ref, g2_ref, bt2_ref, out_ref):
    nf = jnp.float32(N)
    mu = st2_ref[0:1, :] / nf
    var = st2_ref[1:2, :] / nf - mu * mu
    rstd = lax.rsqrt(var + EPS)
    scale = g2_ref[...] * rstd
    shift = bt2_ref[...] - mu * scale
    out_ref[...] = jnp.maximum(h2_ref[...] * scale + shift, 0.0)


def _tc_norm2(h2, st2, g2, bt2):
    return pl.pallas_call(
        _norm2_body,
        grid=(NB,),
        in_specs=[
            pl.BlockSpec((RB, D), lambda i: (i, 0)),
            pl.BlockSpec((2, D), lambda i: (0, 0)),
            pl.BlockSpec((1, D), lambda i: (0, 0)),
            pl.BlockSpec((1, D), lambda i: (0, 0)),
        ],
        out_specs=pl.BlockSpec((RB, D), lambda i: (i, 0)),
        out_shape=jax.ShapeDtypeStruct((N, D), jnp.float32),
    )(h2, st2, g2, bt2)


def kernel(x, rdf_feat, bdf_feat, atom_idx, ele_idx,
           W1r, b1r, g1r, bt1r,
           W1b, b1b, g1b, bt1b,
           W2, b2, g2, bt2):
    aidx = atom_idx.astype(jnp.int32)
    eidx = ele_idx.astype(jnp.int32)

    Wx = jnp.concatenate(
        [jnp.concatenate([W1r[0:256], W1r[512:768]], axis=0),
         jnp.concatenate([W1b[0:256], W1b[512:768]], axis=0)], axis=1)
    Wdr = W1r[1024:1152]
    Wdb = W1b[1024:1152]
    Wa = jnp.concatenate([W1r[256:512], W1b[256:512]], axis=1)
    We = jnp.concatenate([W1r[768:1024], W1b[768:1024]], axis=1)
    bA = jnp.concatenate([b1r, b1b])[None, :]
    g1 = jnp.concatenate([g1r, g1b])[None, :]
    bt1 = jnp.concatenate([bt1r, bt1b])[None, :]

    aidx3 = aidx.reshape(NB, 1, RB)
    psums, pcnt = _tc_pool(x, aidx3)
    T = _tc_tables(psums, pcnt, Wa, We, bA)
    Eg = _sc_gather(T, eidx)
    H0 = _tc_h0a(x, rdf_feat, bdf_feat, aidx3, Wx, Wdr, Wdb, T)
    st1 = _tc_stats1(H0, Eg)
    h2, st2 = _tc_layer2(H0, Eg, st1, g1, bt1, W2, b2[None, :])
    return _tc_norm2(h2, st2, g2[None, :], bt2[None, :])


# speedup vs baseline: 4.1859x; 1.0258x over previous
import functools

import jax
import jax.numpy as jnp
from jax import lax
from jax.experimental import pallas as pl
from jax.experimental.pallas import tpu as pltpu
from jax.experimental.pallas import tpu_sc as plsc

N = 100000
S = 1000
SP = 1024
D = 512
DH = 1024
NC, NS, NW = 2, 16, 32
CH = 80
NCHUNK = N // CH
RB = 1000
NB = N // RB
EPS = 1e-5


GCH = 80
NCH2 = N // GCH
TROWS = 2048

def _gather_body(T_hbm, idx2_hbm, Egi_hbm,
                 buf0, buf1, idxb0, idxb1,
                 semi0, semi1, semg0, semg1, semw0, semw1):
    c = lax.axis_index("c")
    s = lax.axis_index("s")
    wid = s * NC + c
    c0 = (wid * NCH2) // NW
    c1 = ((wid + 1) * NCH2) // NW
    nmine = c1 - c0

    def start_i(i, idxb, sem):
        pltpu.async_copy(idx2_hbm.at[pl.ds((c0 + i) * GCH, GCH)], idxb, sem)

    def wait_i(idxb, sem):
        pltpu.make_async_copy(idx2_hbm.at[pl.ds(0, GCH)], idxb, sem).wait()

    def start_g(buf, idxb, sem):
        pltpu.async_copy(T_hbm.at[idxb], buf, sem)

    def wait_g(buf, sem):
        pltpu.make_async_copy(T_hbm.at[pl.ds(0, GCH)], buf, sem).wait()

    def write_out(i, buf, sem):
        we = pltpu.async_copy(buf, Egi_hbm.at[pl.ds((c0 + i) * GCH, GCH)], sem)
        return we

    start_i(0, idxb0, semi0)
    wait_i(idxb0, semi0)
    start_g(buf0, idxb0, semg0)
    start_i(1, idxb1, semi1)

    def body(k, carry):
        a = 2 * k
        b = a + 1
        wait_g(buf0, semg0)

        @pl.when(b < nmine)
        def _():
            wait_i(idxb1, semi1)
            start_g(buf1, idxb1, semg1)

        wa = write_out(a, buf0, semw0)

        @pl.when(a + 2 < nmine)
        def _():
            start_i(a + 2, idxb0, semi0)

        @pl.when(b < nmine)
        def _():
            wait_g(buf1, semg1)

        wa.wait()

        @pl.when(a + 2 < nmine)
        def _():
            wait_i(idxb0, semi0)
            start_g(buf0, idxb0, semg0)

        @pl.when(b < nmine)
        def _():
            wb = write_out(b, buf1, semw1)

            @pl.when(b + 2 < nmine)
            def _():
                start_i(b + 2, idxb1, semi1)

            wb.wait()

        return carry

    lax.fori_loop(0, (nmine + 1) // 2, body, 0)


def _sc_gather(T, eidx):
    idx2 = eidx + 1024
    mesh = plsc.VectorSubcoreMesh(core_axis_name="c", subcore_axis_name="s")
    f = pl.kernel(
        _gather_body,
        out_type=jax.ShapeDtypeStruct((N, DH // 2), jnp.int32),
        mesh=mesh,
        scratch_types=[
            pltpu.VMEM((GCH, DH // 2), jnp.int32),
            pltpu.VMEM((GCH, DH // 2), jnp.int32),
            pltpu.VMEM((GCH,), jnp.int32),
            pltpu.VMEM((GCH,), jnp.int32),
            pltpu.SemaphoreType.DMA,
            pltpu.SemaphoreType.DMA,
            pltpu.SemaphoreType.DMA,
            pltpu.SemaphoreType.DMA,
            pltpu.SemaphoreType.DMA,
            pltpu.SemaphoreType.DMA,
        ],
    )
    return f(T, idx2)


def _pack16(v_lo, v_hi):
    blo = lax.bitcast_convert_type(
        v_lo.astype(jnp.bfloat16).astype(jnp.float32), jnp.uint32) >> 16
    bhi = lax.bitcast_convert_type(
        v_hi.astype(jnp.bfloat16).astype(jnp.float32), jnp.uint32) >> 16
    return lax.bitcast_convert_type((bhi << 16) | blo, jnp.int32)


def _unpack16(x):
    u = lax.bitcast_convert_type(x, jnp.uint32)
    lo = lax.bitcast_convert_type(u << 16, jnp.float32)
    hi = lax.bitcast_convert_type(u & jnp.uint32(0xFFFF0000), jnp.float32)
    return lo, hi


def _tables_body(ps_ref, pc_ref, Wa_ref, We_ref, bA_ref, T_ref):
    sums = ps_ref[0:S, :]
    cnt = pc_ref[0:S, 0:1].astype(jnp.float32)
    pooled = sums / jnp.maximum(cnt, 1.0)
    pa = pooled[:, 0:256]
    pe = pooled[:, 256:512]
    A = jnp.dot(pa, Wa_ref[...], preferred_element_type=jnp.float32) + bA_ref[...]
    E = jnp.dot(pe, We_ref[...], preferred_element_type=jnp.float32)
    T_ref[0:S, :] = _pack16(A[:, 0:512], A[:, 512:1024])
    T_ref[1024:1024 + S, :] = _pack16(E[:, 0:512], E[:, 512:1024])


def _tc_tables(psums, pcnt, Wa, We, bA):
    return pl.pallas_call(
        _tables_body,
        out_shape=jax.ShapeDtypeStruct((TROWS, DH // 2), jnp.int32),
    )(psums, pcnt, Wa, We, bA)


def _pool_body(x_ref, idx_ref, ps_ref, pc_ref):
    xb16 = x_ref[...].astype(jnp.bfloat16)
    ids = idx_ref[0]
    ohT = (lax.broadcasted_iota(jnp.int32, (SP, RB), 0)
           == jnp.broadcast_to(ids, (SP, RB))).astype(jnp.bfloat16)
    ps = jnp.dot(ohT, xb16, preferred_element_type=jnp.float32)
    pc = jnp.dot(ohT, jnp.ones((RB, 8), jnp.bfloat16),
                 preferred_element_type=jnp.float32)

    @pl.when(pl.program_id(0) == 0)
    def _():
        ps_ref[...] = ps
        pc_ref[...] = pc

    @pl.when(pl.program_id(0) != 0)
    def _():
        ps_ref[...] = ps_ref[...] + ps
        pc_ref[...] = pc_ref[...] + pc


def _tc_pool(x, aidx3):
    return pl.pallas_call(
        _pool_body,
        grid=(NB,),
        in_specs=[
            pl.BlockSpec((RB, D), lambda i: (i, 0)),
            pl.BlockSpec((1, 1, RB), lambda i: (i, 0, 0)),
        ],
        out_specs=(pl.BlockSpec((SP, D), lambda i: (0, 0)),
                   pl.BlockSpec((SP, 8), lambda i: (0, 0))),
        out_shape=(jax.ShapeDtypeStruct((SP, D), jnp.float32),
                   jax.ShapeDtypeStruct((SP, 8), jnp.float32)),
    )(x, aidx3)


def _h0a_body(x_ref, rdf_ref, bdf_ref, idx_ref, Wx_ref, Wdr_ref, Wdb_ref,
              T_ref, H0_ref):
    xb16 = x_ref[...].astype(jnp.bfloat16)
    o = jnp.dot(xb16, Wx_ref[...].astype(jnp.bfloat16),
                preferred_element_type=jnp.float32)
    dr = jnp.dot(rdf_ref[...].astype(jnp.bfloat16),
                 Wdr_ref[...].astype(jnp.bfloat16),
                 preferred_element_type=jnp.float32)
    db = jnp.dot(bdf_ref[...].astype(jnp.bfloat16),
                 Wdb_ref[...].astype(jnp.bfloat16),
                 preferred_element_type=jnp.float32)
    alo, ahi = _unpack16(T_ref[...])
    Abf = jnp.concatenate([alo, ahi], axis=1).astype(jnp.bfloat16)
    ids = idx_ref[0]
    ohT = (lax.broadcasted_iota(jnp.int32, (SP, RB), 0)
           == jnp.broadcast_to(ids, (SP, RB))).astype(jnp.bfloat16)
    ag = lax.dot_general(ohT, Abf, (((0,), (0,)), ((), ())),
                         preferred_element_type=jnp.float32)
    H0_ref[...] = (o + ag
                   + jnp.concatenate([dr, db], axis=1)).astype(jnp.bfloat16)


def _tc_h0a(x, rdf, bdf, aidx3, Wx, Wdr, Wdb, T):
    return pl.pallas_call(
        _h0a_body,
        grid=(NB,),
        in_specs=[
            pl.BlockSpec((RB, D), lambda i: (i, 0)),
            pl.BlockSpec((RB, 128), lambda i: (i, 0)),
            pl.BlockSpec((RB, 128), lambda i: (i, 0)),
            pl.BlockSpec((1, 1, RB), lambda i: (i, 0, 0)),
            pl.BlockSpec((D, DH), lambda i: (0, 0)),
            pl.BlockSpec((128, D), lambda i: (0, 0)),
            pl.BlockSpec((128, D), lambda i: (0, 0)),
            pl.BlockSpec((SP, DH // 2), lambda i: (0, 0)),
        ],
        out_specs=pl.BlockSpec((RB, DH), lambda i: (i, 0)),
        out_shape=jax.ShapeDtypeStruct((N, DH), jnp.bfloat16),
    )(x, rdf, bdf, aidx3, Wx, Wdr, Wdb, T)


def _stats1_body(H0_ref, Eg_ref, st_ref):
    elo, ehi = _unpack16(Eg_ref[...])
    h = H0_ref[...].astype(jnp.float32) + jnp.concatenate([elo, ehi], axis=1)
    ssum = jnp.sum(h, axis=0, keepdims=True)
    sqsum = jnp.sum(h * h, axis=0, keepdims=True)
    blk = jnp.concatenate([ssum, sqsum], axis=0)

    @pl.when(pl.program_id(0) == 0)
    def _():
        st_ref[...] = blk

    @pl.when(pl.program_id(0) != 0)
    def _():
        st_ref[...] = st_ref[...] + blk


def _tc_stats1(H0, Eg):
    return pl.pallas_call(
        _stats1_body,
        grid=(NB,),
        in_specs=[
            pl.BlockSpec((RB, DH), lambda i: (i, 0)),
            pl.BlockSpec((RB, DH // 2), lambda i: (i, 0)),
        ],
        out_specs=pl.BlockSpec((2, DH), lambda i: (0, 0)),
        out_shape=jax.ShapeDtypeStruct((2, DH), jnp.float32),
    )(H0, Eg)


def _layer2_body(H0_ref, Eg_ref, st_ref, g1_ref, bt1_ref,
                 W2_ref, b2_ref, h2_ref, st2_ref):
    nf = jnp.float32(N)
    mu = st_ref[0:1, :] / nf
    var = st_ref[1:2, :] / nf - mu * mu
    rstd = lax.rsqrt(var + EPS)
    scale = g1_ref[...] * rstd
    shift = bt1_ref[...] - mu * scale
    elo, ehi = _unpack16(Eg_ref[...])
    h1 = (H0_ref[...].astype(jnp.float32)
          + jnp.concatenate([elo, ehi], axis=1))
    x12 = jnp.maximum(h1 * scale + shift, 0.0)
    h2 = jnp.dot(x12.astype(jnp.bfloat16), W2_ref[...].astype(jnp.bfloat16),
                 preferred_element_type=jnp.float32) + b2_ref[...]
    h2_ref[...] = h2.astype(jnp.bfloat16)
    ssum = jnp.sum(h2, axis=0, keepdims=True)
    sqsum = jnp.sum(h2 * h2, axis=0, keepdims=True)
    blk = jnp.concatenate([ssum, sqsum], axis=0)

    @pl.when(pl.program_id(0) == 0)
    def _():
        st2_ref[...] = blk

    @pl.when(pl.program_id(0) != 0)
    def _():
        st2_ref[...] = st2_ref[...] + blk


def _tc_layer2(H0, Eg, st1, g1, bt1, W2, b2):
    return pl.pallas_call(
        _layer2_body,
        grid=(NB,),
        in_specs=[
            pl.BlockSpec((RB, DH), lambda i: (i, 0)),
            pl.BlockSpec((RB, DH // 2), lambda i: (i, 0)),
            pl.BlockSpec((2, DH), lambda i: (0, 0)),
            pl.BlockSpec((1, DH), lambda i: (0, 0)),
            pl.BlockSpec((1, DH), lambda i: (0, 0)),
            pl.BlockSpec((DH, D), lambda i: (0, 0)),
            pl.BlockSpec((1, D), lambda i: (0, 0)),
        ],
        out_specs=(pl.BlockSpec((RB, D), lambda i: (i, 0)),
                   pl.BlockSpec((2, D), lambda i: (0, 0))),
        out_shape=(jax.ShapeDtypeStruct((N, D), jnp.bfloat16),
                   jax.ShapeDtypeStruct((2, D), jnp.float32)),
    )(H0, Eg, st1, g1, bt1, W2, b2)


def _norm2_body(h2_ref, st2_ref, g2_ref, bt2_ref, out_ref):
    nf = jnp.float32(N)
    mu = st2_ref[0:1, :] / nf
    var = st2_ref[1:2, :] / nf - mu * mu
    rstd = lax.rsqrt(var + EPS)
    scale = g2_ref[...] * rstd
    shift = bt2_ref[...] - mu * scale
    out_ref[...] = jnp.maximum(h2_ref[...].astype(jnp.float32) * scale + shift,
                               0.0)


def _tc_norm2(h2, st2, g2, bt2):
    return pl.pallas_call(
        _norm2_body,
        grid=(NB,),
        in_specs=[
            pl.BlockSpec((RB, D), lambda i: (i, 0)),
            pl.BlockSpec((2, D), lambda i: (0, 0)),
            pl.BlockSpec((1, D), lambda i: (0, 0)),
            pl.BlockSpec((1, D), lambda i: (0, 0)),
        ],
        out_specs=pl.BlockSpec((RB, D), lambda i: (i, 0)),
        out_shape=jax.ShapeDtypeStruct((N, D), jnp.float32),
    )(h2, st2, g2, bt2)


def kernel(x, rdf_feat, bdf_feat, atom_idx, ele_idx,
           W1r, b1r, g1r, bt1r,
           W1b, b1b, g1b, bt1b,
           W2, b2, g2, bt2):
    aidx = atom_idx.astype(jnp.int32)
    eidx = ele_idx.astype(jnp.int32)

    Wx = jnp.concatenate(
        [jnp.concatenate([W1r[0:256], W1r[512:768]], axis=0),
         jnp.concatenate([W1b[0:256], W1b[512:768]], axis=0)], axis=1)
    Wdr = W1r[1024:1152]
    Wdb = W1b[1024:1152]
    Wa = jnp.concatenate([W1r[256:512], W1b[256:512]], axis=1)
    We = jnp.concatenate([W1r[768:1024], W1b[768:1024]], axis=1)
    bA = jnp.concatenate([b1r, b1b])[None, :]
    g1 = jnp.concatenate([g1r, g1b])[None, :]
    bt1 = jnp.concatenate([bt1r, bt1b])[None, :]

    aidx3 = aidx.reshape(NB, 1, RB)
    psums, pcnt = _tc_pool(x, aidx3)
    T = _tc_tables(psums, pcnt, Wa, We, bA)
    Eg = _sc_gather(T, eidx)
    H0 = _tc_h0a(x, rdf_feat, bdf_feat, aidx3, Wx, Wdr, Wdb, T)
    st1 = _tc_stats1(H0, Eg)
    h2, st2 = _tc_layer2(H0, Eg, st1, g1, bt1, W2, b2[None, :])
    return _tc_norm2(h2, st2, g2[None, :], bt2[None, :])


# speedup vs baseline: 4.1934x; 1.0018x over previous
import functools

import jax
import jax.numpy as jnp
from jax import lax
from jax.experimental import pallas as pl
from jax.experimental.pallas import tpu as pltpu
from jax.experimental.pallas import tpu_sc as plsc

N = 100000
S = 1000
SP = 1024
D = 512
DH = 1024
NC, NS, NW = 2, 16, 32
CH = 80
NCHUNK = N // CH
RB = 1000
NB = N // RB
EPS = 1e-5


GCH = 80
NCH2 = N // GCH
TROWS = 2048

def _gather_body(T_hbm, idx2_hbm, Egi_hbm,
                 buf0, buf1, idxb0, idxb1,
                 semi0, semi1, semg0, semg1, semw0, semw1):
    c = lax.axis_index("c")
    s = lax.axis_index("s")
    wid = s * NC + c
    c0 = (wid * NCH2) // NW
    c1 = ((wid + 1) * NCH2) // NW
    nmine = c1 - c0

    def start_i(i, idxb, sem):
        pltpu.async_copy(idx2_hbm.at[pl.ds((c0 + i) * GCH, GCH)], idxb, sem)

    def wait_i(idxb, sem):
        pltpu.make_async_copy(idx2_hbm.at[pl.ds(0, GCH)], idxb, sem).wait()

    def start_g(buf, idxb, sem):
        pltpu.async_copy(T_hbm.at[idxb], buf, sem)

    def wait_g(buf, sem):
        pltpu.make_async_copy(T_hbm.at[pl.ds(0, GCH)], buf, sem).wait()

    def write_out(i, buf, sem):
        we = pltpu.async_copy(buf, Egi_hbm.at[pl.ds((c0 + i) * GCH, GCH)], sem)
        return we

    start_i(0, idxb0, semi0)
    wait_i(idxb0, semi0)
    start_g(buf0, idxb0, semg0)
    start_i(1, idxb1, semi1)

    def body(k, carry):
        a = 2 * k
        b = a + 1
        wait_g(buf0, semg0)

        @pl.when(b < nmine)
        def _():
            wait_i(idxb1, semi1)
            start_g(buf1, idxb1, semg1)

        wa = write_out(a, buf0, semw0)

        @pl.when(a + 2 < nmine)
        def _():
            start_i(a + 2, idxb0, semi0)

        @pl.when(b < nmine)
        def _():
            wait_g(buf1, semg1)

        wa.wait()

        @pl.when(a + 2 < nmine)
        def _():
            wait_i(idxb0, semi0)
            start_g(buf0, idxb0, semg0)

        @pl.when(b < nmine)
        def _():
            wb = write_out(b, buf1, semw1)

            @pl.when(b + 2 < nmine)
            def _():
                start_i(b + 2, idxb1, semi1)

            wb.wait()

        return carry

    lax.fori_loop(0, (nmine + 1) // 2, body, 0)


def _sc_gather(TE, eidx):
    idx2 = eidx
    mesh = plsc.VectorSubcoreMesh(core_axis_name="c", subcore_axis_name="s")
    f = pl.kernel(
        _gather_body,
        out_type=jax.ShapeDtypeStruct((N, DH // 2), jnp.int32),
        mesh=mesh,
        scratch_types=[
            pltpu.VMEM((GCH, DH // 2), jnp.int32),
            pltpu.VMEM((GCH, DH // 2), jnp.int32),
            pltpu.VMEM((GCH,), jnp.int32),
            pltpu.VMEM((GCH,), jnp.int32),
            pltpu.SemaphoreType.DMA,
            pltpu.SemaphoreType.DMA,
            pltpu.SemaphoreType.DMA,
            pltpu.SemaphoreType.DMA,
            pltpu.SemaphoreType.DMA,
            pltpu.SemaphoreType.DMA,
        ],
    )
    return f(TE, idx2)


def _pack16(v_lo, v_hi):
    blo = lax.bitcast_convert_type(
        v_lo.astype(jnp.bfloat16).astype(jnp.float32), jnp.uint32) >> 16
    bhi = lax.bitcast_convert_type(
        v_hi.astype(jnp.bfloat16).astype(jnp.float32), jnp.uint32) >> 16
    return lax.bitcast_convert_type((bhi << 16) | blo, jnp.int32)


def _unpack16(x):
    u = lax.bitcast_convert_type(x, jnp.uint32)
    lo = lax.bitcast_convert_type(u << 16, jnp.float32)
    hi = lax.bitcast_convert_type(u & jnp.uint32(0xFFFF0000), jnp.float32)
    return lo, hi


def _tables_body(ps_ref, pc_ref, Wa_ref, We_ref, bA_ref, TE_ref, Abf_ref):
    sums = ps_ref[0:S, :]
    cnt = pc_ref[0:S, 0:1].astype(jnp.float32)
    pooled = sums / jnp.maximum(cnt, 1.0)
    pa = pooled[:, 0:256]
    pe = pooled[:, 256:512]
    A = jnp.dot(pa, Wa_ref[...], preferred_element_type=jnp.float32) + bA_ref[...]
    E = jnp.dot(pe, We_ref[...], preferred_element_type=jnp.float32)
    TE_ref[0:S, :] = _pack16(E[:, 0:512], E[:, 512:1024])
    Abf_ref[0:S, :] = A.astype(jnp.bfloat16)
    Abf_ref[S:SP, :] = jnp.zeros((SP - S, DH), jnp.bfloat16)


def _tc_tables(psums, pcnt, Wa, We, bA):
    return pl.pallas_call(
        _tables_body,
        out_shape=(jax.ShapeDtypeStruct((SP, DH // 2), jnp.int32),
                   jax.ShapeDtypeStruct((SP, DH), jnp.bfloat16)),
    )(psums, pcnt, Wa, We, bA)


def _pool_body(x_ref, idx_ref, ps_ref, pc_ref):
    xb16 = x_ref[...].astype(jnp.bfloat16)
    ids = idx_ref[0]
    ohT = (lax.broadcasted_iota(jnp.int32, (SP, RB), 0)
           == jnp.broadcast_to(ids, (SP, RB))).astype(jnp.bfloat16)
    ps = jnp.dot(ohT, xb16, preferred_element_type=jnp.float32)
    pc = jnp.dot(ohT, jnp.ones((RB, 8), jnp.bfloat16),
                 preferred_element_type=jnp.float32)

    @pl.when(pl.program_id(0) == 0)
    def _():
        ps_ref[...] = ps
        pc_ref[...] = pc

    @pl.when(pl.program_id(0) != 0)
    def _():
        ps_ref[...] = ps_ref[...] + ps
        pc_ref[...] = pc_ref[...] + pc


def _tc_pool(x, aidx3):
    return pl.pallas_call(
        _pool_body,
        grid=(NB,),
        in_specs=[
            pl.BlockSpec((RB, D), lambda i: (i, 0)),
            pl.BlockSpec((1, 1, RB), lambda i: (i, 0, 0)),
        ],
        out_specs=(pl.BlockSpec((SP, D), lambda i: (0, 0)),
                   pl.BlockSpec((SP, 8), lambda i: (0, 0))),
        out_shape=(jax.ShapeDtypeStruct((SP, D), jnp.float32),
                   jax.ShapeDtypeStruct((SP, 8), jnp.float32)),
    )(x, aidx3)


def _h0a_body(x_ref, rdf_ref, bdf_ref, idx_ref, Wx_ref, Wdr_ref, Wdb_ref,
              Abf_ref, H0_ref):
    xb16 = x_ref[...].astype(jnp.bfloat16)
    o = jnp.dot(xb16, Wx_ref[...], preferred_element_type=jnp.float32)
    dr = jnp.dot(rdf_ref[...].astype(jnp.bfloat16), Wdr_ref[...],
                 preferred_element_type=jnp.float32)
    db = jnp.dot(bdf_ref[...].astype(jnp.bfloat16), Wdb_ref[...],
                 preferred_element_type=jnp.float32)
    ids = idx_ref[0]
    ohT = (lax.broadcasted_iota(jnp.int32, (SP, RB), 0)
           == jnp.broadcast_to(ids, (SP, RB))).astype(jnp.bfloat16)
    ag = lax.dot_general(ohT, Abf_ref[...], (((0,), (0,)), ((), ())),
                         preferred_element_type=jnp.float32)
    H0_ref[...] = (o + ag
                   + jnp.concatenate([dr, db], axis=1)).astype(jnp.bfloat16)


def _tc_h0a(x, rdf, bdf, aidx3, Wx, Wdr, Wdb, Abf):
    return pl.pallas_call(
        _h0a_body,
        grid=(NB,),
        in_specs=[
            pl.BlockSpec((RB, D), lambda i: (i, 0)),
            pl.BlockSpec((RB, 128), lambda i: (i, 0)),
            pl.BlockSpec((RB, 128), lambda i: (i, 0)),
            pl.BlockSpec((1, 1, RB), lambda i: (i, 0, 0)),
            pl.BlockSpec((D, DH), lambda i: (0, 0)),
            pl.BlockSpec((128, D), lambda i: (0, 0)),
            pl.BlockSpec((128, D), lambda i: (0, 0)),
            pl.BlockSpec((SP, DH), lambda i: (0, 0)),
        ],
        out_specs=pl.BlockSpec((RB, DH), lambda i: (i, 0)),
        out_shape=jax.ShapeDtypeStruct((N, DH), jnp.bfloat16),
    )(x, rdf, bdf, aidx3, Wx, Wdr, Wdb, Abf)


def _stats1_body(H0_ref, Eg_ref, st_ref):
    elo, ehi = _unpack16(Eg_ref[...])
    h = H0_ref[...].astype(jnp.float32) + jnp.concatenate([elo, ehi], axis=1)
    ssum = jnp.sum(h, axis=0, keepdims=True)
    sqsum = jnp.sum(h * h, axis=0, keepdims=True)
    blk = jnp.concatenate([ssum, sqsum], axis=0)

    @pl.when(pl.program_id(0) == 0)
    def _():
        st_ref[...] = blk

    @pl.when(pl.program_id(0) != 0)
    def _():
        st_ref[...] = st_ref[...] + blk


def _tc_stats1(H0, Eg):
    return pl.pallas_call(
        _stats1_body,
        grid=(NB,),
        in_specs=[
            pl.BlockSpec((RB, DH), lambda i: (i, 0)),
            pl.BlockSpec((RB, DH // 2), lambda i: (i, 0)),
        ],
        out_specs=pl.BlockSpec((2, DH), lambda i: (0, 0)),
        out_shape=jax.ShapeDtypeStruct((2, DH), jnp.float32),
    )(H0, Eg)


def _layer2_body(H0_ref, Eg_ref, st_ref, g1_ref, bt1_ref,
                 W2_ref, b2_ref, h2_ref, st2_ref):
    nf = jnp.float32(N)
    mu = st_ref[0:1, :] / nf
    var = st_ref[1:2, :] / nf - mu * mu
    rstd = lax.rsqrt(var + EPS)
    scale = g1_ref[...] * rstd
    shift = bt1_ref[...] - mu * scale
    elo, ehi = _unpack16(Eg_ref[...])
    h1 = (H0_ref[...].astype(jnp.float32)
          + jnp.concatenate([elo, ehi], axis=1))
    x12 = jnp.maximum(h1 * scale + shift, 0.0)
    h2 = jnp.dot(x12.astype(jnp.bfloat16), W2_ref[...],
                 preferred_element_type=jnp.float32) + b2_ref[...]
    h2_ref[...] = h2.astype(jnp.bfloat16)
    ssum = jnp.sum(h2, axis=0, keepdims=True)
    sqsum = jnp.sum(h2 * h2, axis=0, keepdims=True)
    blk = jnp.concatenate([ssum, sqsum], axis=0)

    @pl.when(pl.program_id(0) == 0)
    def _():
        st2_ref[...] = blk

    @pl.when(pl.program_id(0) != 0)
    def _():
        st2_ref[...] = st2_ref[...] + blk


def _tc_layer2(H0, Eg, st1, g1, bt1, W2, b2):
    return pl.pallas_call(
        _layer2_body,
        grid=(NB,),
        in_specs=[
            pl.BlockSpec((RB, DH), lambda i: (i, 0)),
            pl.BlockSpec((RB, DH // 2), lambda i: (i, 0)),
            pl.BlockSpec((2, DH), lambda i: (0, 0)),
            pl.BlockSpec((1, DH), lambda i: (0, 0)),
            pl.BlockSpec((1, DH), lambda i: (0, 0)),
            pl.BlockSpec((DH, D), lambda i: (0, 0)),
            pl.BlockSpec((1, D), lambda i: (0, 0)),
        ],
        out_specs=(pl.BlockSpec((RB, D), lambda i: (i, 0)),
                   pl.BlockSpec((2, D), lambda i: (0, 0))),
        out_shape=(jax.ShapeDtypeStruct((N, D), jnp.bfloat16),
                   jax.ShapeDtypeStruct((2, D), jnp.float32)),
    )(H0, Eg, st1, g1, bt1, W2, b2)


def _norm2_body(h2_ref, st2_ref, g2_ref, bt2_ref, out_ref):
    nf = jnp.float32(N)
    mu = st2_ref[0:1, :] / nf
    var = st2_ref[1:2, :] / nf - mu * mu
    rstd = lax.rsqrt(var + EPS)
    scale = g2_ref[...] * rstd
    shift = bt2_ref[...] - mu * scale
    out_ref[...] = jnp.maximum(h2_ref[...].astype(jnp.float32) * scale + shift,
                               0.0)


def _tc_norm2(h2, st2, g2, bt2):
    return pl.pallas_call(
        _norm2_body,
        grid=(NB,),
        in_specs=[
            pl.BlockSpec((RB, D), lambda i: (i, 0)),
            pl.BlockSpec((2, D), lambda i: (0, 0)),
            pl.BlockSpec((1, D), lambda i: (0, 0)),
            pl.BlockSpec((1, D), lambda i: (0, 0)),
        ],
        out_specs=pl.BlockSpec((RB, D), lambda i: (i, 0)),
        out_shape=jax.ShapeDtypeStruct((N, D), jnp.float32),
    )(h2, st2, g2, bt2)


def kernel(x, rdf_feat, bdf_feat, atom_idx, ele_idx,
           W1r, b1r, g1r, bt1r,
           W1b, b1b, g1b, bt1b,
           W2, b2, g2, bt2):
    aidx = atom_idx.astype(jnp.int32)
    eidx = ele_idx.astype(jnp.int32)

    Wx = jnp.concatenate(
        [jnp.concatenate([W1r[0:256], W1r[512:768]], axis=0),
         jnp.concatenate([W1b[0:256], W1b[512:768]], axis=0)], axis=1)
    Wdr = W1r[1024:1152]
    Wdb = W1b[1024:1152]
    Wa = jnp.concatenate([W1r[256:512], W1b[256:512]], axis=1)
    We = jnp.concatenate([W1r[768:1024], W1b[768:1024]], axis=1)
    bA = jnp.concatenate([b1r, b1b])[None, :]
    g1 = jnp.concatenate([g1r, g1b])[None, :]
    bt1 = jnp.concatenate([bt1r, bt1b])[None, :]

    aidx3 = aidx.reshape(NB, 1, RB)
    psums, pcnt = _tc_pool(x, aidx3)
    TE, Abf = _tc_tables(psums, pcnt, Wa, We, bA)
    Eg = _sc_gather(TE, eidx)
    H0 = _tc_h0a(x, rdf_feat, bdf_feat, aidx3,
                 Wx.astype(jnp.bfloat16), Wdr.astype(jnp.bfloat16),
                 Wdb.astype(jnp.bfloat16), Abf)
    st1 = _tc_stats1(H0, Eg)
    h2, st2 = _tc_layer2(H0, Eg, st1, g1, bt1,
                         W2.astype(jnp.bfloat16), b2[None, :])
    return _tc_norm2(h2, st2, g2[None, :], bt2[None, :])


# speedup vs baseline: 4.2641x; 1.0169x over previous
import functools

import jax
import jax.numpy as jnp
from jax import lax
from jax.experimental import pallas as pl
from jax.experimental.pallas import tpu as pltpu
from jax.experimental.pallas import tpu_sc as plsc

N = 100000
S = 1000
SP = 1024
D = 512
DH = 1024
NC, NS, NW = 2, 16, 32
CH = 80
NCHUNK = N // CH
RB = 1000
NB = N // RB
EPS = 1e-5


GCH = 80
NCH2 = N // GCH
TROWS = 2048

def _gather_body(T_hbm, idx2_hbm, Egi_hbm,
                 buf0, buf1, idxb0, idxb1,
                 semi0, semi1, semg0, semg1, semw0, semw1):
    c = lax.axis_index("c")
    s = lax.axis_index("s")
    wid = s * NC + c
    c0 = (wid * NCH2) // NW
    c1 = ((wid + 1) * NCH2) // NW
    nmine = c1 - c0

    def start_i(i, idxb, sem):
        pltpu.async_copy(idx2_hbm.at[pl.ds((c0 + i) * GCH, GCH)], idxb, sem)

    def wait_i(idxb, sem):
        pltpu.make_async_copy(idx2_hbm.at[pl.ds(0, GCH)], idxb, sem).wait()

    def start_g(buf, idxb, sem):
        pltpu.async_copy(T_hbm.at[idxb], buf, sem)

    def wait_g(buf, sem):
        pltpu.make_async_copy(T_hbm.at[pl.ds(0, GCH)], buf, sem).wait()

    def write_out(i, buf, sem):
        we = pltpu.async_copy(buf, Egi_hbm.at[pl.ds((c0 + i) * GCH, GCH)], sem)
        return we

    start_i(0, idxb0, semi0)
    wait_i(idxb0, semi0)
    start_g(buf0, idxb0, semg0)
    start_i(1, idxb1, semi1)

    def body(k, carry):
        a = 2 * k
        b = a + 1
        wait_g(buf0, semg0)

        @pl.when(b < nmine)
        def _():
            wait_i(idxb1, semi1)
            start_g(buf1, idxb1, semg1)

        wa = write_out(a, buf0, semw0)

        @pl.when(a + 2 < nmine)
        def _():
            start_i(a + 2, idxb0, semi0)

        @pl.when(b < nmine)
        def _():
            wait_g(buf1, semg1)

        wa.wait()

        @pl.when(a + 2 < nmine)
        def _():
            wait_i(idxb0, semi0)
            start_g(buf0, idxb0, semg0)

        @pl.when(b < nmine)
        def _():
            wb = write_out(b, buf1, semw1)

            @pl.when(b + 2 < nmine)
            def _():
                start_i(b + 2, idxb1, semi1)

            wb.wait()

        return carry

    lax.fori_loop(0, (nmine + 1) // 2, body, 0)


def _sc_gather(TE, eidx):
    idx2 = eidx
    mesh = plsc.VectorSubcoreMesh(core_axis_name="c", subcore_axis_name="s")
    f = pl.kernel(
        _gather_body,
        out_type=jax.ShapeDtypeStruct((N, DH // 2), jnp.int32),
        mesh=mesh,
        scratch_types=[
            pltpu.VMEM((GCH, DH // 2), jnp.int32),
            pltpu.VMEM((GCH, DH // 2), jnp.int32),
            pltpu.VMEM((GCH,), jnp.int32),
            pltpu.VMEM((GCH,), jnp.int32),
            pltpu.SemaphoreType.DMA,
            pltpu.SemaphoreType.DMA,
            pltpu.SemaphoreType.DMA,
            pltpu.SemaphoreType.DMA,
            pltpu.SemaphoreType.DMA,
            pltpu.SemaphoreType.DMA,
        ],
    )
    return f(TE, idx2)


def _pack16(v_lo, v_hi):
    blo = lax.bitcast_convert_type(
        v_lo.astype(jnp.bfloat16).astype(jnp.float32), jnp.uint32) >> 16
    bhi = lax.bitcast_convert_type(
        v_hi.astype(jnp.bfloat16).astype(jnp.float32), jnp.uint32) >> 16
    return lax.bitcast_convert_type((bhi << 16) | blo, jnp.int32)


def _unpack16(x):
    u = lax.bitcast_convert_type(x, jnp.uint32)
    lo = lax.bitcast_convert_type(u << 16, jnp.float32)
    hi = lax.bitcast_convert_type(u & jnp.uint32(0xFFFF0000), jnp.float32)
    return lo, hi


def _tables_body(ps_ref, pc_ref, Wa_ref, We_ref, bA_ref, TE_ref, Abf_ref):
    sums = ps_ref[0:S, :]
    cnt = pc_ref[0:S, 0:1].astype(jnp.float32)
    pooled = sums / jnp.maximum(cnt, 1.0)
    pa = pooled[:, 0:256]
    pe = pooled[:, 256:512]
    A = jnp.dot(pa, Wa_ref[...], preferred_element_type=jnp.float32) + bA_ref[...]
    E = jnp.dot(pe, We_ref[...], preferred_element_type=jnp.float32)
    TE_ref[0:S, :] = _pack16(E[:, 0:512], E[:, 512:1024])
    Abf_ref[0:S, :] = A.astype(jnp.bfloat16)
    Abf_ref[S:SP, :] = jnp.zeros((SP - S, DH), jnp.bfloat16)


def _tc_tables(psums, pcnt, Wa, We, bA):
    return pl.pallas_call(
        _tables_body,
        out_shape=(jax.ShapeDtypeStruct((SP, DH // 2), jnp.int32),
                   jax.ShapeDtypeStruct((SP, DH), jnp.bfloat16)),
    )(psums, pcnt, Wa, We, bA)


RP = 2000
NBP = N // RP

def _pool_body(x_ref, idx_ref, ps_ref, pc_ref):
    xb16 = x_ref[...].astype(jnp.bfloat16)
    ids = idx_ref[0]
    ohT = (lax.broadcasted_iota(jnp.int32, (SP, RP), 0)
           == jnp.broadcast_to(ids, (SP, RP))).astype(jnp.bfloat16)
    ps = jnp.dot(ohT, xb16, preferred_element_type=jnp.float32)
    pc = jnp.dot(ohT, jnp.ones((RP, 8), jnp.bfloat16),
                 preferred_element_type=jnp.float32)

    @pl.when(pl.program_id(0) == 0)
    def _():
        ps_ref[...] = ps
        pc_ref[...] = pc

    @pl.when(pl.program_id(0) != 0)
    def _():
        ps_ref[...] = ps_ref[...] + ps
        pc_ref[...] = pc_ref[...] + pc


def _tc_pool(x, aidx3p):
    return pl.pallas_call(
        _pool_body,
        grid=(NBP,),
        in_specs=[
            pl.BlockSpec((RP, D), lambda i: (i, 0)),
            pl.BlockSpec((1, 1, RP), lambda i: (i, 0, 0)),
        ],
        out_specs=(pl.BlockSpec((SP, D), lambda i: (0, 0)),
                   pl.BlockSpec((SP, 8), lambda i: (0, 0))),
        out_shape=(jax.ShapeDtypeStruct((SP, D), jnp.float32),
                   jax.ShapeDtypeStruct((SP, 8), jnp.float32)),
    )(x, aidx3p)


def _h0a_body(x_ref, rdf_ref, bdf_ref, idx_ref, Wx_ref, Wdr_ref, Wdb_ref,
              Abf_ref, H0_ref):
    xb16 = x_ref[...].astype(jnp.bfloat16)
    o = jnp.dot(xb16, Wx_ref[...], preferred_element_type=jnp.float32)
    dr = jnp.dot(rdf_ref[...].astype(jnp.bfloat16), Wdr_ref[...],
                 preferred_element_type=jnp.float32)
    db = jnp.dot(bdf_ref[...].astype(jnp.bfloat16), Wdb_ref[...],
                 preferred_element_type=jnp.float32)
    ids = idx_ref[0]
    ohT = (lax.broadcasted_iota(jnp.int32, (SP, RB), 0)
           == jnp.broadcast_to(ids, (SP, RB))).astype(jnp.bfloat16)
    ag = lax.dot_general(ohT, Abf_ref[...], (((0,), (0,)), ((), ())),
                         preferred_element_type=jnp.float32)
    H0_ref[...] = (o + ag
                   + jnp.concatenate([dr, db], axis=1)).astype(jnp.bfloat16)


def _tc_h0a(x, rdf, bdf, aidx3, Wx, Wdr, Wdb, Abf):
    return pl.pallas_call(
        _h0a_body,
        grid=(NB,),
        in_specs=[
            pl.BlockSpec((RB, D), lambda i: (i, 0)),
            pl.BlockSpec((RB, 128), lambda i: (i, 0)),
            pl.BlockSpec((RB, 128), lambda i: (i, 0)),
            pl.BlockSpec((1, 1, RB), lambda i: (i, 0, 0)),
            pl.BlockSpec((D, DH), lambda i: (0, 0)),
            pl.BlockSpec((128, D), lambda i: (0, 0)),
            pl.BlockSpec((128, D), lambda i: (0, 0)),
            pl.BlockSpec((SP, DH), lambda i: (0, 0)),
        ],
        out_specs=pl.BlockSpec((RB, DH), lambda i: (i, 0)),
        out_shape=jax.ShapeDtypeStruct((N, DH), jnp.bfloat16),
    )(x, rdf, bdf, aidx3, Wx, Wdr, Wdb, Abf)


def _stats1_body(H0_ref, Eg_ref, st_ref):
    elo, ehi = _unpack16(Eg_ref[...])
    h = H0_ref[...].astype(jnp.float32) + jnp.concatenate([elo, ehi], axis=1)
    ssum = jnp.sum(h, axis=0, keepdims=True)
    sqsum = jnp.sum(h * h, axis=0, keepdims=True)
    blk = jnp.concatenate([ssum, sqsum], axis=0)

    @pl.when(pl.program_id(0) == 0)
    def _():
        st_ref[...] = blk

    @pl.when(pl.program_id(0) != 0)
    def _():
        st_ref[...] = st_ref[...] + blk


def _tc_stats1(H0, Eg):
    return pl.pallas_call(
        _stats1_body,
        grid=(NB,),
        in_specs=[
            pl.BlockSpec((RB, DH), lambda i: (i, 0)),
            pl.BlockSpec((RB, DH // 2), lambda i: (i, 0)),
        ],
        out_specs=pl.BlockSpec((2, DH), lambda i: (0, 0)),
        out_shape=jax.ShapeDtypeStruct((2, DH), jnp.float32),
    )(H0, Eg)


def _layer2_body(H0_ref, Eg_ref, st_ref, g1_ref, bt1_ref,
                 W2_ref, b2_ref, h2_ref, st2_ref):
    nf = jnp.float32(N)
    mu = st_ref[0:1, :] / nf
    var = st_ref[1:2, :] / nf - mu * mu
    rstd = lax.rsqrt(var + EPS)
    scale = g1_ref[...] * rstd
    shift = bt1_ref[...] - mu * scale
    elo, ehi = _unpack16(Eg_ref[...])
    h1 = (H0_ref[...].astype(jnp.float32)
          + jnp.concatenate([elo, ehi], axis=1))
    x12 = jnp.maximum(h1 * scale + shift, 0.0)
    h2 = jnp.dot(x12.astype(jnp.bfloat16), W2_ref[...],
                 preferred_element_type=jnp.float32) + b2_ref[...]
    h2_ref[...] = h2.astype(jnp.bfloat16)
    ssum = jnp.sum(h2, axis=0, keepdims=True)
    sqsum = jnp.sum(h2 * h2, axis=0, keepdims=True)
    blk = jnp.concatenate([ssum, sqsum], axis=0)

    @pl.when(pl.program_id(0) == 0)
    def _():
        st2_ref[...] = blk

    @pl.when(pl.program_id(0) != 0)
    def _():
        st2_ref[...] = st2_ref[...] + blk


def _tc_layer2(H0, Eg, st1, g1, bt1, W2, b2):
    return pl.pallas_call(
        _layer2_body,
        grid=(NB,),
        in_specs=[
            pl.BlockSpec((RB, DH), lambda i: (i, 0)),
            pl.BlockSpec((RB, DH // 2), lambda i: (i, 0)),
            pl.BlockSpec((2, DH), lambda i: (0, 0)),
            pl.BlockSpec((1, DH), lambda i: (0, 0)),
            pl.BlockSpec((1, DH), lambda i: (0, 0)),
            pl.BlockSpec((DH, D), lambda i: (0, 0)),
            pl.BlockSpec((1, D), lambda i: (0, 0)),
        ],
        out_specs=(pl.BlockSpec((RB, D), lambda i: (i, 0)),
                   pl.BlockSpec((2, D), lambda i: (0, 0))),
        out_shape=(jax.ShapeDtypeStruct((N, D), jnp.bfloat16),
                   jax.ShapeDtypeStruct((2, D), jnp.float32)),
    )(H0, Eg, st1, g1, bt1, W2, b2)


def _norm2_body(h2_ref, st2_ref, g2_ref, bt2_ref, out_ref):
    nf = jnp.float32(N)
    mu = st2_ref[0:1, :] / nf
    var = st2_ref[1:2, :] / nf - mu * mu
    rstd = lax.rsqrt(var + EPS)
    scale = g2_ref[...] * rstd
    shift = bt2_ref[...] - mu * scale
    out_ref[...] = jnp.maximum(h2_ref[...].astype(jnp.float32) * scale + shift,
                               0.0)


def _tc_norm2(h2, st2, g2, bt2):
    return pl.pallas_call(
        _norm2_body,
        grid=(NB,),
        in_specs=[
            pl.BlockSpec((RB, D), lambda i: (i, 0)),
            pl.BlockSpec((2, D), lambda i: (0, 0)),
            pl.BlockSpec((1, D), lambda i: (0, 0)),
            pl.BlockSpec((1, D), lambda i: (0, 0)),
        ],
        out_specs=pl.BlockSpec((RB, D), lambda i: (i, 0)),
        out_shape=jax.ShapeDtypeStruct((N, D), jnp.float32),
    )(h2, st2, g2, bt2)


def kernel(x, rdf_feat, bdf_feat, atom_idx, ele_idx,
           W1r, b1r, g1r, bt1r,
           W1b, b1b, g1b, bt1b,
           W2, b2, g2, bt2):
    aidx = atom_idx.astype(jnp.int32)
    eidx = ele_idx.astype(jnp.int32)

    Wx = jnp.concatenate(
        [jnp.concatenate([W1r[0:256], W1r[512:768]], axis=0),
         jnp.concatenate([W1b[0:256], W1b[512:768]], axis=0)], axis=1)
    Wdr = W1r[1024:1152]
    Wdb = W1b[1024:1152]
    Wa = jnp.concatenate([W1r[256:512], W1b[256:512]], axis=1)
    We = jnp.concatenate([W1r[768:1024], W1b[768:1024]], axis=1)
    bA = jnp.concatenate([b1r, b1b])[None, :]
    g1 = jnp.concatenate([g1r, g1b])[None, :]
    bt1 = jnp.concatenate([bt1r, bt1b])[None, :]

    aidx3 = aidx.reshape(NB, 1, RB)
    psums, pcnt = _tc_pool(x, aidx.reshape(NBP, 1, RP))
    TE, Abf = _tc_tables(psums, pcnt, Wa, We, bA)
    Eg = _sc_gather(TE, eidx)
    H0 = _tc_h0a(x, rdf_feat, bdf_feat, aidx3,
                 Wx.astype(jnp.bfloat16), Wdr.astype(jnp.bfloat16),
                 Wdb.astype(jnp.bfloat16), Abf)
    st1 = _tc_stats1(H0, Eg)
    h2, st2 = _tc_layer2(H0, Eg, st1, g1, bt1,
                         W2.astype(jnp.bfloat16), b2[None, :])
    return _tc_norm2(h2, st2, g2[None, :], bt2[None, :])


# speedup vs baseline: 4.6613x; 1.0932x over previous
import functools

import jax
import jax.numpy as jnp
from jax import lax
from jax.experimental import pallas as pl
from jax.experimental.pallas import tpu as pltpu
from jax.experimental.pallas import tpu_sc as plsc

N = 100000
S = 1000
SP = 1024
D = 512
DH = 1024
NC, NS, NW = 2, 16, 32
CH = 80
NCHUNK = N // CH
RB = 2000
NB = N // RB
EPS = 1e-5


GCH = 80
NCH2 = N // GCH
TROWS = 2048

def _gather_body(T_hbm, idx2_hbm, Egi_hbm,
                 buf0, buf1, idxb0, idxb1,
                 semi0, semi1, semg0, semg1, semw0, semw1):
    c = lax.axis_index("c")
    s = lax.axis_index("s")
    wid = s * NC + c
    c0 = (wid * NCH2) // NW
    c1 = ((wid + 1) * NCH2) // NW
    nmine = c1 - c0

    def start_i(i, idxb, sem):
        pltpu.async_copy(idx2_hbm.at[pl.ds((c0 + i) * GCH, GCH)], idxb, sem)

    def wait_i(idxb, sem):
        pltpu.make_async_copy(idx2_hbm.at[pl.ds(0, GCH)], idxb, sem).wait()

    def start_g(buf, idxb, sem):
        pltpu.async_copy(T_hbm.at[idxb], buf, sem)

    def wait_g(buf, sem):
        pltpu.make_async_copy(T_hbm.at[pl.ds(0, GCH)], buf, sem).wait()

    def write_out(i, buf, sem):
        we = pltpu.async_copy(buf, Egi_hbm.at[pl.ds((c0 + i) * GCH, GCH)], sem)
        return we

    start_i(0, idxb0, semi0)
    wait_i(idxb0, semi0)
    start_g(buf0, idxb0, semg0)
    start_i(1, idxb1, semi1)

    def body(k, carry):
        a = 2 * k
        b = a + 1
        wait_g(buf0, semg0)

        @pl.when(b < nmine)
        def _():
            wait_i(idxb1, semi1)
            start_g(buf1, idxb1, semg1)

        wa = write_out(a, buf0, semw0)

        @pl.when(a + 2 < nmine)
        def _():
            start_i(a + 2, idxb0, semi0)

        @pl.when(b < nmine)
        def _():
            wait_g(buf1, semg1)

        wa.wait()

        @pl.when(a + 2 < nmine)
        def _():
            wait_i(idxb0, semi0)
            start_g(buf0, idxb0, semg0)

        @pl.when(b < nmine)
        def _():
            wb = write_out(b, buf1, semw1)

            @pl.when(b + 2 < nmine)
            def _():
                start_i(b + 2, idxb1, semi1)

            wb.wait()

        return carry

    lax.fori_loop(0, (nmine + 1) // 2, body, 0)


def _sc_gather(TE, eidx):
    idx2 = eidx
    mesh = plsc.VectorSubcoreMesh(core_axis_name="c", subcore_axis_name="s")
    f = pl.kernel(
        _gather_body,
        out_type=jax.ShapeDtypeStruct((N, DH // 2), jnp.int32),
        mesh=mesh,
        scratch_types=[
            pltpu.VMEM((GCH, DH // 2), jnp.int32),
            pltpu.VMEM((GCH, DH // 2), jnp.int32),
            pltpu.VMEM((GCH,), jnp.int32),
            pltpu.VMEM((GCH,), jnp.int32),
            pltpu.SemaphoreType.DMA,
            pltpu.SemaphoreType.DMA,
            pltpu.SemaphoreType.DMA,
            pltpu.SemaphoreType.DMA,
            pltpu.SemaphoreType.DMA,
            pltpu.SemaphoreType.DMA,
        ],
    )
    return f(TE, idx2)


def _pack16(v_lo, v_hi):
    blo = lax.bitcast_convert_type(
        v_lo.astype(jnp.bfloat16).astype(jnp.float32), jnp.uint32) >> 16
    bhi = lax.bitcast_convert_type(
        v_hi.astype(jnp.bfloat16).astype(jnp.float32), jnp.uint32) >> 16
    return lax.bitcast_convert_type((bhi << 16) | blo, jnp.int32)


def _unpack16(x):
    u = lax.bitcast_convert_type(x, jnp.uint32)
    lo = lax.bitcast_convert_type(u << 16, jnp.float32)
    hi = lax.bitcast_convert_type(u & jnp.uint32(0xFFFF0000), jnp.float32)
    return lo, hi


def _tables_body(ps_ref, pc_ref, Wa_ref, We_ref, bA_ref, TE_ref, Abf_ref):
    sums = ps_ref[0:S, :]
    cnt = pc_ref[0:S, 0:1].astype(jnp.float32)
    pooled = sums / jnp.maximum(cnt, 1.0)
    pa = pooled[:, 0:256]
    pe = pooled[:, 256:512]
    A = jnp.dot(pa, Wa_ref[...], preferred_element_type=jnp.float32) + bA_ref[...]
    E = jnp.dot(pe, We_ref[...], preferred_element_type=jnp.float32)
    TE_ref[0:S, :] = _pack16(E[:, 0:512], E[:, 512:1024])
    Abf_ref[0:S, :] = A.astype(jnp.bfloat16)
    Abf_ref[S:SP, :] = jnp.zeros((SP - S, DH), jnp.bfloat16)


def _tc_tables(psums, pcnt, Wa, We, bA):
    return pl.pallas_call(
        _tables_body,
        out_shape=(jax.ShapeDtypeStruct((SP, DH // 2), jnp.int32),
                   jax.ShapeDtypeStruct((SP, DH), jnp.bfloat16)),
    )(psums, pcnt, Wa, We, bA)


RP = 2000
NBP = N // RP

def _pool_body(x_ref, idx_ref, ps_ref, pc_ref):
    xb16 = x_ref[...].astype(jnp.bfloat16)
    ids = idx_ref[0]
    ohT = (lax.broadcasted_iota(jnp.int32, (SP, RP), 0)
           == jnp.broadcast_to(ids, (SP, RP))).astype(jnp.bfloat16)
    ps = jnp.dot(ohT, xb16, preferred_element_type=jnp.float32)
    pc = jnp.dot(ohT, jnp.ones((RP, 8), jnp.bfloat16),
                 preferred_element_type=jnp.float32)

    @pl.when(pl.program_id(0) == 0)
    def _():
        ps_ref[...] = ps
        pc_ref[...] = pc

    @pl.when(pl.program_id(0) != 0)
    def _():
        ps_ref[...] = ps_ref[...] + ps
        pc_ref[...] = pc_ref[...] + pc


def _tc_pool(x, aidx3p):
    return pl.pallas_call(
        _pool_body,
        grid=(NBP,),
        in_specs=[
            pl.BlockSpec((RP, D), lambda i: (i, 0)),
            pl.BlockSpec((1, 1, RP), lambda i: (i, 0, 0)),
        ],
        out_specs=(pl.BlockSpec((SP, D), lambda i: (0, 0)),
                   pl.BlockSpec((SP, 8), lambda i: (0, 0))),
        out_shape=(jax.ShapeDtypeStruct((SP, D), jnp.float32),
                   jax.ShapeDtypeStruct((SP, 8), jnp.float32)),
    )(x, aidx3p)


def _h0a_body(x_ref, rdf_ref, bdf_ref, idx_ref, Wx_ref, Wdr_ref, Wdb_ref,
              Abf_ref, H0_ref):
    xb16 = x_ref[...].astype(jnp.bfloat16)
    o = jnp.dot(xb16, Wx_ref[...], preferred_element_type=jnp.float32)
    dr = jnp.dot(rdf_ref[...].astype(jnp.bfloat16), Wdr_ref[...],
                 preferred_element_type=jnp.float32)
    db = jnp.dot(bdf_ref[...].astype(jnp.bfloat16), Wdb_ref[...],
                 preferred_element_type=jnp.float32)
    ids = idx_ref[0]
    ohT = (lax.broadcasted_iota(jnp.int32, (SP, RB), 0)
           == jnp.broadcast_to(ids, (SP, RB))).astype(jnp.bfloat16)
    ag = lax.dot_general(ohT, Abf_ref[...], (((0,), (0,)), ((), ())),
                         preferred_element_type=jnp.float32)
    H0_ref[...] = (o + ag
                   + jnp.concatenate([dr, db], axis=1)).astype(jnp.bfloat16)


def _tc_h0a(x, rdf, bdf, aidx3, Wx, Wdr, Wdb, Abf):
    return pl.pallas_call(
        _h0a_body,
        grid=(NB,),
        in_specs=[
            pl.BlockSpec((RB, D), lambda i: (i, 0)),
            pl.BlockSpec((RB, 128), lambda i: (i, 0)),
            pl.BlockSpec((RB, 128), lambda i: (i, 0)),
            pl.BlockSpec((1, 1, RB), lambda i: (i, 0, 0)),
            pl.BlockSpec((D, DH), lambda i: (0, 0)),
            pl.BlockSpec((128, D), lambda i: (0, 0)),
            pl.BlockSpec((128, D), lambda i: (0, 0)),
            pl.BlockSpec((SP, DH), lambda i: (0, 0)),
        ],
        out_specs=pl.BlockSpec((RB, DH), lambda i: (i, 0)),
        out_shape=jax.ShapeDtypeStruct((N, DH), jnp.bfloat16),
    )(x, rdf, bdf, aidx3, Wx, Wdr, Wdb, Abf)


def _stats1_body(H0_ref, Eg_ref, st_ref):
    elo, ehi = _unpack16(Eg_ref[...])
    h = H0_ref[...].astype(jnp.float32) + jnp.concatenate([elo, ehi], axis=1)
    ssum = jnp.sum(h, axis=0, keepdims=True)
    sqsum = jnp.sum(h * h, axis=0, keepdims=True)
    blk = jnp.concatenate([ssum, sqsum], axis=0)

    @pl.when(pl.program_id(0) == 0)
    def _():
        st_ref[...] = blk

    @pl.when(pl.program_id(0) != 0)
    def _():
        st_ref[...] = st_ref[...] + blk


def _tc_stats1(H0, Eg):
    return pl.pallas_call(
        _stats1_body,
        grid=(NB,),
        in_specs=[
            pl.BlockSpec((RB, DH), lambda i: (i, 0)),
            pl.BlockSpec((RB, DH // 2), lambda i: (i, 0)),
        ],
        out_specs=pl.BlockSpec((2, DH), lambda i: (0, 0)),
        out_shape=jax.ShapeDtypeStruct((2, DH), jnp.float32),
    )(H0, Eg)


def _layer2_body(H0_ref, Eg_ref, st_ref, g1_ref, bt1_ref,
                 W2_ref, b2_ref, h2_ref, st2_ref):
    nf = jnp.float32(N)
    mu = st_ref[0:1, :] / nf
    var = st_ref[1:2, :] / nf - mu * mu
    rstd = lax.rsqrt(var + EPS)
    scale = g1_ref[...] * rstd
    shift = bt1_ref[...] - mu * scale
    elo, ehi = _unpack16(Eg_ref[...])
    h1 = (H0_ref[...].astype(jnp.float32)
          + jnp.concatenate([elo, ehi], axis=1))
    x12 = jnp.maximum(h1 * scale + shift, 0.0)
    h2 = jnp.dot(x12.astype(jnp.bfloat16), W2_ref[...],
                 preferred_element_type=jnp.float32) + b2_ref[...]
    h2_ref[...] = h2.astype(jnp.bfloat16)
    ssum = jnp.sum(h2, axis=0, keepdims=True)
    sqsum = jnp.sum(h2 * h2, axis=0, keepdims=True)
    blk = jnp.concatenate([ssum, sqsum], axis=0)

    @pl.when(pl.program_id(0) == 0)
    def _():
        st2_ref[...] = blk

    @pl.when(pl.program_id(0) != 0)
    def _():
        st2_ref[...] = st2_ref[...] + blk


def _tc_layer2(H0, Eg, st1, g1, bt1, W2, b2):
    return pl.pallas_call(
        _layer2_body,
        grid=(NB,),
        in_specs=[
            pl.BlockSpec((RB, DH), lambda i: (i, 0)),
            pl.BlockSpec((RB, DH // 2), lambda i: (i, 0)),
            pl.BlockSpec((2, DH), lambda i: (0, 0)),
            pl.BlockSpec((1, DH), lambda i: (0, 0)),
            pl.BlockSpec((1, DH), lambda i: (0, 0)),
            pl.BlockSpec((DH, D), lambda i: (0, 0)),
            pl.BlockSpec((1, D), lambda i: (0, 0)),
        ],
        out_specs=(pl.BlockSpec((RB, D), lambda i: (i, 0)),
                   pl.BlockSpec((2, D), lambda i: (0, 0))),
        out_shape=(jax.ShapeDtypeStruct((N, D), jnp.bfloat16),
                   jax.ShapeDtypeStruct((2, D), jnp.float32)),
    )(H0, Eg, st1, g1, bt1, W2, b2)


def _norm2_body(h2_ref, st2_ref, g2_ref, bt2_ref, out_ref):
    nf = jnp.float32(N)
    mu = st2_ref[0:1, :] / nf
    var = st2_ref[1:2, :] / nf - mu * mu
    rstd = lax.rsqrt(var + EPS)
    scale = g2_ref[...] * rstd
    shift = bt2_ref[...] - mu * scale
    out_ref[...] = jnp.maximum(h2_ref[...].astype(jnp.float32) * scale + shift,
                               0.0)


def _tc_norm2(h2, st2, g2, bt2):
    return pl.pallas_call(
        _norm2_body,
        grid=(NB,),
        in_specs=[
            pl.BlockSpec((RB, D), lambda i: (i, 0)),
            pl.BlockSpec((2, D), lambda i: (0, 0)),
            pl.BlockSpec((1, D), lambda i: (0, 0)),
            pl.BlockSpec((1, D), lambda i: (0, 0)),
        ],
        out_specs=pl.BlockSpec((RB, D), lambda i: (i, 0)),
        out_shape=jax.ShapeDtypeStruct((N, D), jnp.float32),
    )(h2, st2, g2, bt2)


def kernel(x, rdf_feat, bdf_feat, atom_idx, ele_idx,
           W1r, b1r, g1r, bt1r,
           W1b, b1b, g1b, bt1b,
           W2, b2, g2, bt2):
    aidx = atom_idx.astype(jnp.int32)
    eidx = ele_idx.astype(jnp.int32)

    Wx = jnp.concatenate(
        [jnp.concatenate([W1r[0:256], W1r[512:768]], axis=0),
         jnp.concatenate([W1b[0:256], W1b[512:768]], axis=0)], axis=1)
    Wdr = W1r[1024:1152]
    Wdb = W1b[1024:1152]
    Wa = jnp.concatenate([W1r[256:512], W1b[256:512]], axis=1)
    We = jnp.concatenate([W1r[768:1024], W1b[768:1024]], axis=1)
    bA = jnp.concatenate([b1r, b1b])[None, :]
    g1 = jnp.concatenate([g1r, g1b])[None, :]
    bt1 = jnp.concatenate([bt1r, bt1b])[None, :]

    aidx3 = aidx.reshape(NB, 1, RB)
    psums, pcnt = _tc_pool(x, aidx.reshape(NBP, 1, RP))
    TE, Abf = _tc_tables(psums, pcnt, Wa, We, bA)
    Eg = _sc_gather(TE, eidx)
    H0 = _tc_h0a(x, rdf_feat, bdf_feat, aidx3,
                 Wx.astype(jnp.bfloat16), Wdr.astype(jnp.bfloat16),
                 Wdb.astype(jnp.bfloat16), Abf)
    st1 = _tc_stats1(H0, Eg)
    h2, st2 = _tc_layer2(H0, Eg, st1, g1, bt1,
                         W2.astype(jnp.bfloat16), b2[None, :])
    return _tc_norm2(h2, st2, g2[None, :], bt2[None, :])


# speedup vs baseline: 4.6616x; 1.0001x over previous
import jax
import jax.numpy as jnp
from jax import lax
from jax.experimental import pallas as pl
from jax.experimental.pallas import tpu as pltpu
from jax.experimental.pallas import tpu_sc as plsc

N = 100000
S = 1000
SP = 1024
D = 512
DH = 1024
NC, NS, NW = 2, 16, 32
CH = 80
NCHUNK = N // CH
RB = 2000
NB = N // RB
EPS = 1e-5


GCH = 80
NCH2 = N // GCH

def _gather_body(T_hbm, idx2_hbm, Egi_hbm,
                 buf0, buf1, idxb0, idxb1,
                 semi0, semi1, semg0, semg1, semw0, semw1):
    c = lax.axis_index("c")
    s = lax.axis_index("s")
    wid = s * NC + c
    c0 = (wid * NCH2) // NW
    c1 = ((wid + 1) * NCH2) // NW
    nmine = c1 - c0

    def start_i(i, idxb, sem):
        pltpu.async_copy(idx2_hbm.at[pl.ds((c0 + i) * GCH, GCH)], idxb, sem)

    def wait_i(idxb, sem):
        pltpu.make_async_copy(idx2_hbm.at[pl.ds(0, GCH)], idxb, sem).wait()

    def start_g(buf, idxb, sem):
        pltpu.async_copy(T_hbm.at[idxb], buf, sem)

    def wait_g(buf, sem):
        pltpu.make_async_copy(T_hbm.at[pl.ds(0, GCH)], buf, sem).wait()

    def write_out(i, buf, sem):
        we = pltpu.async_copy(buf, Egi_hbm.at[pl.ds((c0 + i) * GCH, GCH)], sem)
        return we

    start_i(0, idxb0, semi0)
    wait_i(idxb0, semi0)
    start_g(buf0, idxb0, semg0)
    start_i(1, idxb1, semi1)

    def body(k, carry):
        a = 2 * k
        b = a + 1
        wait_g(buf0, semg0)

        @pl.when(b < nmine)
        def _():
            wait_i(idxb1, semi1)
            start_g(buf1, idxb1, semg1)

        wa = write_out(a, buf0, semw0)

        @pl.when(a + 2 < nmine)
        def _():
            start_i(a + 2, idxb0, semi0)

        @pl.when(b < nmine)
        def _():
            wait_g(buf1, semg1)

        wa.wait()

        @pl.when(a + 2 < nmine)
        def _():
            wait_i(idxb0, semi0)
            start_g(buf0, idxb0, semg0)

        @pl.when(b < nmine)
        def _():
            wb = write_out(b, buf1, semw1)

            @pl.when(b + 2 < nmine)
            def _():
                start_i(b + 2, idxb1, semi1)

            wb.wait()

        return carry

    lax.fori_loop(0, (nmine + 1) // 2, body, 0)


def _sc_gather(TE, eidx):
    idx2 = eidx
    mesh = plsc.VectorSubcoreMesh(core_axis_name="c", subcore_axis_name="s")
    f = pl.kernel(
        _gather_body,
        out_type=jax.ShapeDtypeStruct((N, DH // 2), jnp.int32),
        mesh=mesh,
        scratch_types=[
            pltpu.VMEM((GCH, DH // 2), jnp.int32),
            pltpu.VMEM((GCH, DH // 2), jnp.int32),
            pltpu.VMEM((GCH,), jnp.int32),
            pltpu.VMEM((GCH,), jnp.int32),
            pltpu.SemaphoreType.DMA,
            pltpu.SemaphoreType.DMA,
            pltpu.SemaphoreType.DMA,
            pltpu.SemaphoreType.DMA,
            pltpu.SemaphoreType.DMA,
            pltpu.SemaphoreType.DMA,
        ],
    )
    return f(TE, idx2)


def _pack16(v_lo, v_hi):
    blo = lax.bitcast_convert_type(
        v_lo.astype(jnp.bfloat16).astype(jnp.float32), jnp.uint32) >> 16
    bhi = lax.bitcast_convert_type(
        v_hi.astype(jnp.bfloat16).astype(jnp.float32), jnp.uint32) >> 16
    return lax.bitcast_convert_type((bhi << 16) | blo, jnp.int32)


def _unpack16(x):
    u = lax.bitcast_convert_type(x, jnp.uint32)
    lo = lax.bitcast_convert_type(u << 16, jnp.float32)
    hi = lax.bitcast_convert_type(u & jnp.uint32(0xFFFF0000), jnp.float32)
    return lo, hi


def _tables_body(ps_ref, pc_ref, Wa_ref, We_ref, bA_ref, TE_ref, Abf_ref):
    sums = ps_ref[0:S, :]
    cnt = pc_ref[0:S, 0:1].astype(jnp.float32)
    pooled = sums / jnp.maximum(cnt, 1.0)
    pa = pooled[:, 0:256]
    pe = pooled[:, 256:512]
    A = jnp.dot(pa, Wa_ref[...], preferred_element_type=jnp.float32) + bA_ref[...]
    E = jnp.dot(pe, We_ref[...], preferred_element_type=jnp.float32)
    TE_ref[0:S, :] = _pack16(E[:, 0:512], E[:, 512:1024])
    Abf_ref[0:S, :] = A.astype(jnp.bfloat16)
    Abf_ref[S:SP, :] = jnp.zeros((SP - S, DH), jnp.bfloat16)


def _tc_tables(psums, pcnt, Wa, We, bA):
    return pl.pallas_call(
        _tables_body,
        out_shape=(jax.ShapeDtypeStruct((SP, DH // 2), jnp.int32),
                   jax.ShapeDtypeStruct((SP, DH), jnp.bfloat16)),
    )(psums, pcnt, Wa, We, bA)


RP = 2000
NBP = N // RP

def _pool_body(x_ref, idx_ref, ps_ref, pc_ref):
    xb16 = x_ref[...].astype(jnp.bfloat16)
    ids = idx_ref[0]
    ohT = (lax.broadcasted_iota(jnp.int32, (SP, RP), 0)
           == jnp.broadcast_to(ids, (SP, RP))).astype(jnp.bfloat16)
    ps = jnp.dot(ohT, xb16, preferred_element_type=jnp.float32)
    pc = jnp.dot(ohT, jnp.ones((RP, 8), jnp.bfloat16),
                 preferred_element_type=jnp.float32)

    @pl.when(pl.program_id(0) == 0)
    def _():
        ps_ref[...] = ps
        pc_ref[...] = pc

    @pl.when(pl.program_id(0) != 0)
    def _():
        ps_ref[...] = ps_ref[...] + ps
        pc_ref[...] = pc_ref[...] + pc


def _tc_pool(x, aidx3p):
    return pl.pallas_call(
        _pool_body,
        grid=(NBP,),
        in_specs=[
            pl.BlockSpec((RP, D), lambda i: (i, 0)),
            pl.BlockSpec((1, 1, RP), lambda i: (i, 0, 0)),
        ],
        out_specs=(pl.BlockSpec((SP, D), lambda i: (0, 0)),
                   pl.BlockSpec((SP, 8), lambda i: (0, 0))),
        out_shape=(jax.ShapeDtypeStruct((SP, D), jnp.float32),
                   jax.ShapeDtypeStruct((SP, 8), jnp.float32)),
    )(x, aidx3p)


def _h0a_body(x_ref, rdf_ref, bdf_ref, idx_ref, Wx_ref, Wdr_ref, Wdb_ref,
              Abf_ref, H0_ref):
    xb16 = x_ref[...].astype(jnp.bfloat16)
    o = jnp.dot(xb16, Wx_ref[...], preferred_element_type=jnp.float32)
    dr = jnp.dot(rdf_ref[...].astype(jnp.bfloat16), Wdr_ref[...],
                 preferred_element_type=jnp.float32)
    db = jnp.dot(bdf_ref[...].astype(jnp.bfloat16), Wdb_ref[...],
                 preferred_element_type=jnp.float32)
    ids = idx_ref[0]
    ohT = (lax.broadcasted_iota(jnp.int32, (SP, RB), 0)
           == jnp.broadcast_to(ids, (SP, RB))).astype(jnp.bfloat16)
    ag = lax.dot_general(ohT, Abf_ref[...], (((0,), (0,)), ((), ())),
                         preferred_element_type=jnp.float32)
    H0_ref[...] = (o + ag
                   + jnp.concatenate([dr, db], axis=1)).astype(jnp.bfloat16)


def _tc_h0a(x, rdf, bdf, aidx3, Wx, Wdr, Wdb, Abf):
    return pl.pallas_call(
        _h0a_body,
        grid=(NB,),
        in_specs=[
            pl.BlockSpec((RB, D), lambda i: (i, 0)),
            pl.BlockSpec((RB, 128), lambda i: (i, 0)),
            pl.BlockSpec((RB, 128), lambda i: (i, 0)),
            pl.BlockSpec((1, 1, RB), lambda i: (i, 0, 0)),
            pl.BlockSpec((D, DH), lambda i: (0, 0)),
            pl.BlockSpec((128, D), lambda i: (0, 0)),
            pl.BlockSpec((128, D), lambda i: (0, 0)),
            pl.BlockSpec((SP, DH), lambda i: (0, 0)),
        ],
        out_specs=pl.BlockSpec((RB, DH), lambda i: (i, 0)),
        out_shape=jax.ShapeDtypeStruct((N, DH), jnp.bfloat16),
    )(x, rdf, bdf, aidx3, Wx, Wdr, Wdb, Abf)


def _stats1_body(H0_ref, Eg_ref, st_ref):
    elo, ehi = _unpack16(Eg_ref[...])
    h = H0_ref[...].astype(jnp.float32) + jnp.concatenate([elo, ehi], axis=1)
    ssum = jnp.sum(h, axis=0, keepdims=True)
    sqsum = jnp.sum(h * h, axis=0, keepdims=True)
    blk = jnp.concatenate([ssum, sqsum], axis=0)

    @pl.when(pl.program_id(0) == 0)
    def _():
        st_ref[...] = blk

    @pl.when(pl.program_id(0) != 0)
    def _():
        st_ref[...] = st_ref[...] + blk


def _tc_stats1(H0, Eg):
    return pl.pallas_call(
        _stats1_body,
        grid=(NB,),
        in_specs=[
            pl.BlockSpec((RB, DH), lambda i: (i, 0)),
            pl.BlockSpec((RB, DH // 2), lambda i: (i, 0)),
        ],
        out_specs=pl.BlockSpec((2, DH), lambda i: (0, 0)),
        out_shape=jax.ShapeDtypeStruct((2, DH), jnp.float32),
    )(H0, Eg)


def _layer2_body(H0_ref, Eg_ref, st_ref, g1_ref, bt1_ref,
                 W2_ref, b2_ref, h2_ref, st2_ref):
    nf = jnp.float32(N)
    mu = st_ref[0:1, :] / nf
    var = st_ref[1:2, :] / nf - mu * mu
    rstd = lax.rsqrt(var + EPS)
    scale = g1_ref[...] * rstd
    shift = bt1_ref[...] - mu * scale
    elo, ehi = _unpack16(Eg_ref[...])
    h1 = (H0_ref[...].astype(jnp.float32)
          + jnp.concatenate([elo, ehi], axis=1))
    x12 = jnp.maximum(h1 * scale + shift, 0.0)
    h2 = jnp.dot(x12.astype(jnp.bfloat16), W2_ref[...],
                 preferred_element_type=jnp.float32) + b2_ref[...]
    h2_ref[...] = h2.astype(jnp.bfloat16)
    ssum = jnp.sum(h2, axis=0, keepdims=True)
    sqsum = jnp.sum(h2 * h2, axis=0, keepdims=True)
    blk = jnp.concatenate([ssum, sqsum], axis=0)

    @pl.when(pl.program_id(0) == 0)
    def _():
        st2_ref[...] = blk

    @pl.when(pl.program_id(0) != 0)
    def _():
        st2_ref[...] = st2_ref[...] + blk


def _tc_layer2(H0, Eg, st1, g1, bt1, W2, b2):
    return pl.pallas_call(
        _layer2_body,
        grid=(NB,),
        in_specs=[
            pl.BlockSpec((RB, DH), lambda i: (i, 0)),
            pl.BlockSpec((RB, DH // 2), lambda i: (i, 0)),
            pl.BlockSpec((2, DH), lambda i: (0, 0)),
            pl.BlockSpec((1, DH), lambda i: (0, 0)),
            pl.BlockSpec((1, DH), lambda i: (0, 0)),
            pl.BlockSpec((DH, D), lambda i: (0, 0)),
            pl.BlockSpec((1, D), lambda i: (0, 0)),
        ],
        out_specs=(pl.BlockSpec((RB, D), lambda i: (i, 0)),
                   pl.BlockSpec((2, D), lambda i: (0, 0))),
        out_shape=(jax.ShapeDtypeStruct((N, D), jnp.bfloat16),
                   jax.ShapeDtypeStruct((2, D), jnp.float32)),
    )(H0, Eg, st1, g1, bt1, W2, b2)


def _norm2_body(h2_ref, st2_ref, g2_ref, bt2_ref, out_ref):
    nf = jnp.float32(N)
    mu = st2_ref[0:1, :] / nf
    var = st2_ref[1:2, :] / nf - mu * mu
    rstd = lax.rsqrt(var + EPS)
    scale = g2_ref[...] * rstd
    shift = bt2_ref[...] - mu * scale
    out_ref[...] = jnp.maximum(h2_ref[...].astype(jnp.float32) * scale + shift,
                               0.0)


def _tc_norm2(h2, st2, g2, bt2):
    return pl.pallas_call(
        _norm2_body,
        grid=(NB,),
        in_specs=[
            pl.BlockSpec((RB, D), lambda i: (i, 0)),
            pl.BlockSpec((2, D), lambda i: (0, 0)),
            pl.BlockSpec((1, D), lambda i: (0, 0)),
            pl.BlockSpec((1, D), lambda i: (0, 0)),
        ],
        out_specs=pl.BlockSpec((RB, D), lambda i: (i, 0)),
        out_shape=jax.ShapeDtypeStruct((N, D), jnp.float32),
    )(h2, st2, g2, bt2)


def kernel(x, rdf_feat, bdf_feat, atom_idx, ele_idx,
           W1r, b1r, g1r, bt1r,
           W1b, b1b, g1b, bt1b,
           W2, b2, g2, bt2):
    aidx = atom_idx.astype(jnp.int32)
    eidx = ele_idx.astype(jnp.int32)

    Wx = jnp.concatenate(
        [jnp.concatenate([W1r[0:256], W1r[512:768]], axis=0),
         jnp.concatenate([W1b[0:256], W1b[512:768]], axis=0)], axis=1)
    Wdr = W1r[1024:1152]
    Wdb = W1b[1024:1152]
    Wa = jnp.concatenate([W1r[256:512], W1b[256:512]], axis=1)
    We = jnp.concatenate([W1r[768:1024], W1b[768:1024]], axis=1)
    bA = jnp.concatenate([b1r, b1b])[None, :]
    g1 = jnp.concatenate([g1r, g1b])[None, :]
    bt1 = jnp.concatenate([bt1r, bt1b])[None, :]

    aidx3 = aidx.reshape(NB, 1, RB)
    psums, pcnt = _tc_pool(x, aidx.reshape(NBP, 1, RP))
    TE, Abf = _tc_tables(psums, pcnt, Wa, We, bA)
    Eg = _sc_gather(TE, eidx)
    H0 = _tc_h0a(x, rdf_feat, bdf_feat, aidx3,
                 Wx.astype(jnp.bfloat16), Wdr.astype(jnp.bfloat16),
                 Wdb.astype(jnp.bfloat16), Abf)
    st1 = _tc_stats1(H0, Eg)
    h2, st2 = _tc_layer2(H0, Eg, st1, g1, bt1,
                         W2.astype(jnp.bfloat16), b2[None, :])
    return _tc_norm2(h2, st2, g2[None, :], bt2[None, :])


# speedup vs baseline: 4.6628x; 1.0003x over previous
import jax
import jax.numpy as jnp
from jax import lax
from jax.experimental import pallas as pl
from jax.experimental.pallas import tpu as pltpu
from jax.experimental.pallas import tpu_sc as plsc

N = 100000
S = 1000
SP = 1024
D = 512
DH = 1024
NC, NW = 2, 32
RB = 2000
NB = N // RB
EPS = 1e-5


GCH = 80
NCH2 = N // GCH

def _gather_body(T_hbm, idx2_hbm, Egi_hbm,
                 buf0, buf1, idxb0, idxb1,
                 semi0, semi1, semg0, semg1, semw0, semw1):
    c = lax.axis_index("c")
    s = lax.axis_index("s")
    wid = s * NC + c
    c0 = (wid * NCH2) // NW
    c1 = ((wid + 1) * NCH2) // NW
    nmine = c1 - c0

    def start_i(i, idxb, sem):
        pltpu.async_copy(idx2_hbm.at[pl.ds((c0 + i) * GCH, GCH)], idxb, sem)

    def wait_i(idxb, sem):
        pltpu.make_async_copy(idx2_hbm.at[pl.ds(0, GCH)], idxb, sem).wait()

    def start_g(buf, idxb, sem):
        pltpu.async_copy(T_hbm.at[idxb], buf, sem)

    def wait_g(buf, sem):
        pltpu.make_async_copy(T_hbm.at[pl.ds(0, GCH)], buf, sem).wait()

    def write_out(i, buf, sem):
        we = pltpu.async_copy(buf, Egi_hbm.at[pl.ds((c0 + i) * GCH, GCH)], sem)
        return we

    start_i(0, idxb0, semi0)
    wait_i(idxb0, semi0)
    start_g(buf0, idxb0, semg0)
    start_i(1, idxb1, semi1)

    def body(k, carry):
        a = 2 * k
        b = a + 1
        wait_g(buf0, semg0)

        @pl.when(b < nmine)
        def _():
            wait_i(idxb1, semi1)
            start_g(buf1, idxb1, semg1)

        wa = write_out(a, buf0, semw0)

        @pl.when(a + 2 < nmine)
        def _():
            start_i(a + 2, idxb0, semi0)

        @pl.when(b < nmine)
        def _():
            wait_g(buf1, semg1)

        wa.wait()

        @pl.when(a + 2 < nmine)
        def _():
            wait_i(idxb0, semi0)
            start_g(buf0, idxb0, semg0)

        @pl.when(b < nmine)
        def _():
            wb = write_out(b, buf1, semw1)

            @pl.when(b + 2 < nmine)
            def _():
                start_i(b + 2, idxb1, semi1)

            wb.wait()

        return carry

    lax.fori_loop(0, (nmine + 1) // 2, body, 0)


def _sc_gather(TE, eidx):
    idx2 = eidx
    mesh = plsc.VectorSubcoreMesh(core_axis_name="c", subcore_axis_name="s")
    f = pl.kernel(
        _gather_body,
        out_type=jax.ShapeDtypeStruct((N, DH // 2), jnp.int32),
        mesh=mesh,
        scratch_types=[
            pltpu.VMEM((GCH, DH // 2), jnp.int32),
            pltpu.VMEM((GCH, DH // 2), jnp.int32),
            pltpu.VMEM((GCH,), jnp.int32),
            pltpu.VMEM((GCH,), jnp.int32),
            pltpu.SemaphoreType.DMA,
            pltpu.SemaphoreType.DMA,
            pltpu.SemaphoreType.DMA,
            pltpu.SemaphoreType.DMA,
            pltpu.SemaphoreType.DMA,
            pltpu.SemaphoreType.DMA,
        ],
    )
    return f(TE, idx2)


def _pack16(v_lo, v_hi):
    blo = lax.bitcast_convert_type(
        v_lo.astype(jnp.bfloat16).astype(jnp.float32), jnp.uint32) >> 16
    bhi = lax.bitcast_convert_type(
        v_hi.astype(jnp.bfloat16).astype(jnp.float32), jnp.uint32) >> 16
    return lax.bitcast_convert_type((bhi << 16) | blo, jnp.int32)


def _unpack16(x):
    u = lax.bitcast_convert_type(x, jnp.uint32)
    lo = lax.bitcast_convert_type(u << 16, jnp.float32)
    hi = lax.bitcast_convert_type(u & jnp.uint32(0xFFFF0000), jnp.float32)
    return lo, hi


def _tables_body(ps_ref, pc_ref, Wa_ref, We_ref, bA_ref, TE_ref, Abf_ref):
    sums = ps_ref[0:S, :]
    cnt = pc_ref[0:S, 0:1].astype(jnp.float32)
    pooled = sums / jnp.maximum(cnt, 1.0)
    pa = pooled[:, 0:256]
    pe = pooled[:, 256:512]
    A = jnp.dot(pa, Wa_ref[...], preferred_element_type=jnp.float32) + bA_ref[...]
    E = jnp.dot(pe, We_ref[...], preferred_element_type=jnp.float32)
    TE_ref[0:S, :] = _pack16(E[:, 0:512], E[:, 512:1024])
    Abf_ref[0:S, :] = A.astype(jnp.bfloat16)
    Abf_ref[S:SP, :] = jnp.zeros((SP - S, DH), jnp.bfloat16)


def _tc_tables(psums, pcnt, Wa, We, bA):
    return pl.pallas_call(
        _tables_body,
        out_shape=(jax.ShapeDtypeStruct((SP, DH // 2), jnp.int32),
                   jax.ShapeDtypeStruct((SP, DH), jnp.bfloat16)),
    )(psums, pcnt, Wa, We, bA)


RP = 2000
NBP = N // RP

def _pool_body(x_ref, idx_ref, ps_ref, pc_ref):
    xb16 = x_ref[...].astype(jnp.bfloat16)
    ids = idx_ref[0]
    ohT = (lax.broadcasted_iota(jnp.int32, (SP, RP), 0)
           == jnp.broadcast_to(ids, (SP, RP))).astype(jnp.bfloat16)
    ps = jnp.dot(ohT, xb16, preferred_element_type=jnp.float32)
    pc = jnp.dot(ohT, jnp.ones((RP, 8), jnp.bfloat16),
                 preferred_element_type=jnp.float32)

    @pl.when(pl.program_id(0) == 0)
    def _():
        ps_ref[...] = ps
        pc_ref[...] = pc

    @pl.when(pl.program_id(0) != 0)
    def _():
        ps_ref[...] = ps_ref[...] + ps
        pc_ref[...] = pc_ref[...] + pc


def _tc_pool(x, aidx3p):
    return pl.pallas_call(
        _pool_body,
        grid=(NBP,),
        in_specs=[
            pl.BlockSpec((RP, D), lambda i: (i, 0)),
            pl.BlockSpec((1, 1, RP), lambda i: (i, 0, 0)),
        ],
        out_specs=(pl.BlockSpec((SP, D), lambda i: (0, 0)),
                   pl.BlockSpec((SP, 8), lambda i: (0, 0))),
        out_shape=(jax.ShapeDtypeStruct((SP, D), jnp.float32),
                   jax.ShapeDtypeStruct((SP, 8), jnp.float32)),
    )(x, aidx3p)


def _h0a_body(x_ref, rdf_ref, bdf_ref, idx_ref, Wx_ref, Wdr_ref, Wdb_ref,
              Abf_ref, H0_ref):
    xb16 = x_ref[...].astype(jnp.bfloat16)
    o = jnp.dot(xb16, Wx_ref[...], preferred_element_type=jnp.float32)
    dr = jnp.dot(rdf_ref[...].astype(jnp.bfloat16), Wdr_ref[...],
                 preferred_element_type=jnp.float32)
    db = jnp.dot(bdf_ref[...].astype(jnp.bfloat16), Wdb_ref[...],
                 preferred_element_type=jnp.float32)
    ids = idx_ref[0]
    ohT = (lax.broadcasted_iota(jnp.int32, (SP, RB), 0)
           == jnp.broadcast_to(ids, (SP, RB))).astype(jnp.bfloat16)
    ag = lax.dot_general(ohT, Abf_ref[...], (((0,), (0,)), ((), ())),
                         preferred_element_type=jnp.float32)
    H0_ref[...] = (o + ag
                   + jnp.concatenate([dr, db], axis=1)).astype(jnp.bfloat16)


def _tc_h0a(x, rdf, bdf, aidx3, Wx, Wdr, Wdb, Abf):
    return pl.pallas_call(
        _h0a_body,
        grid=(NB,),
        in_specs=[
            pl.BlockSpec((RB, D), lambda i: (i, 0)),
            pl.BlockSpec((RB, 128), lambda i: (i, 0)),
            pl.BlockSpec((RB, 128), lambda i: (i, 0)),
            pl.BlockSpec((1, 1, RB), lambda i: (i, 0, 0)),
            pl.BlockSpec((D, DH), lambda i: (0, 0)),
            pl.BlockSpec((128, D), lambda i: (0, 0)),
            pl.BlockSpec((128, D), lambda i: (0, 0)),
            pl.BlockSpec((SP, DH), lambda i: (0, 0)),
        ],
        out_specs=pl.BlockSpec((RB, DH), lambda i: (i, 0)),
        out_shape=jax.ShapeDtypeStruct((N, DH), jnp.bfloat16),
    )(x, rdf, bdf, aidx3, Wx, Wdr, Wdb, Abf)


def _stats1_body(H0_ref, Eg_ref, st_ref):
    elo, ehi = _unpack16(Eg_ref[...])
    h = H0_ref[...].astype(jnp.float32) + jnp.concatenate([elo, ehi], axis=1)
    ssum = jnp.sum(h, axis=0, keepdims=True)
    sqsum = jnp.sum(h * h, axis=0, keepdims=True)
    blk = jnp.concatenate([ssum, sqsum], axis=0)

    @pl.when(pl.program_id(0) == 0)
    def _():
        st_ref[...] = blk

    @pl.when(pl.program_id(0) != 0)
    def _():
        st_ref[...] = st_ref[...] + blk


def _tc_stats1(H0, Eg):
    return pl.pallas_call(
        _stats1_body,
        grid=(NB,),
        in_specs=[
            pl.BlockSpec((RB, DH), lambda i: (i, 0)),
            pl.BlockSpec((RB, DH // 2), lambda i: (i, 0)),
        ],
        out_specs=pl.BlockSpec((2, DH), lambda i: (0, 0)),
        out_shape=jax.ShapeDtypeStruct((2, DH), jnp.float32),
    )(H0, Eg)


def _layer2_body(H0_ref, Eg_ref, st_ref, g1_ref, bt1_ref,
                 W2_ref, b2_ref, h2_ref, st2_ref):
    nf = jnp.float32(N)
    mu = st_ref[0:1, :] / nf
    var = st_ref[1:2, :] / nf - mu * mu
    rstd = lax.rsqrt(var + EPS)
    scale = g1_ref[...] * rstd
    shift = bt1_ref[...] - mu * scale
    elo, ehi = _unpack16(Eg_ref[...])
    h1 = (H0_ref[...].astype(jnp.float32)
          + jnp.concatenate([elo, ehi], axis=1))
    x12 = jnp.maximum(h1 * scale + shift, 0.0)
    h2 = jnp.dot(x12.astype(jnp.bfloat16), W2_ref[...],
                 preferred_element_type=jnp.float32) + b2_ref[...]
    h2_ref[...] = h2.astype(jnp.bfloat16)
    ssum = jnp.sum(h2, axis=0, keepdims=True)
    sqsum = jnp.sum(h2 * h2, axis=0, keepdims=True)
    blk = jnp.concatenate([ssum, sqsum], axis=0)

    @pl.when(pl.program_id(0) == 0)
    def _():
        st2_ref[...] = blk

    @pl.when(pl.program_id(0) != 0)
    def _():
        st2_ref[...] = st2_ref[...] + blk


def _tc_layer2(H0, Eg, st1, g1, bt1, W2, b2):
    return pl.pallas_call(
        _layer2_body,
        grid=(NB,),
        in_specs=[
            pl.BlockSpec((RB, DH), lambda i: (i, 0)),
            pl.BlockSpec((RB, DH // 2), lambda i: (i, 0)),
            pl.BlockSpec((2, DH), lambda i: (0, 0)),
            pl.BlockSpec((1, DH), lambda i: (0, 0)),
            pl.BlockSpec((1, DH), lambda i: (0, 0)),
            pl.BlockSpec((DH, D), lambda i: (0, 0)),
            pl.BlockSpec((1, D), lambda i: (0, 0)),
        ],
        out_specs=(pl.BlockSpec((RB, D), lambda i: (i, 0)),
                   pl.BlockSpec((2, D), lambda i: (0, 0))),
        out_shape=(jax.ShapeDtypeStruct((N, D), jnp.bfloat16),
                   jax.ShapeDtypeStruct((2, D), jnp.float32)),
    )(H0, Eg, st1, g1, bt1, W2, b2)


def _norm2_body(h2_ref, st2_ref, g2_ref, bt2_ref, out_ref):
    nf = jnp.float32(N)
    mu = st2_ref[0:1, :] / nf
    var = st2_ref[1:2, :] / nf - mu * mu
    rstd = lax.rsqrt(var + EPS)
    scale = g2_ref[...] * rstd
    shift = bt2_ref[...] - mu * scale
    out_ref[...] = jnp.maximum(h2_ref[...].astype(jnp.float32) * scale + shift,
                               0.0)


def _tc_norm2(h2, st2, g2, bt2):
    return pl.pallas_call(
        _norm2_body,
        grid=(NB,),
        in_specs=[
            pl.BlockSpec((RB, D), lambda i: (i, 0)),
            pl.BlockSpec((2, D), lambda i: (0, 0)),
            pl.BlockSpec((1, D), lambda i: (0, 0)),
            pl.BlockSpec((1, D), lambda i: (0, 0)),
        ],
        out_specs=pl.BlockSpec((RB, D), lambda i: (i, 0)),
        out_shape=jax.ShapeDtypeStruct((N, D), jnp.float32),
    )(h2, st2, g2, bt2)


def kernel(x, rdf_feat, bdf_feat, atom_idx, ele_idx,
           W1r, b1r, g1r, bt1r,
           W1b, b1b, g1b, bt1b,
           W2, b2, g2, bt2):
    aidx = atom_idx.astype(jnp.int32)
    eidx = ele_idx.astype(jnp.int32)

    Wx = jnp.concatenate(
        [jnp.concatenate([W1r[0:256], W1r[512:768]], axis=0),
         jnp.concatenate([W1b[0:256], W1b[512:768]], axis=0)], axis=1)
    Wdr = W1r[1024:1152]
    Wdb = W1b[1024:1152]
    Wa = jnp.concatenate([W1r[256:512], W1b[256:512]], axis=1)
    We = jnp.concatenate([W1r[768:1024], W1b[768:1024]], axis=1)
    bA = jnp.concatenate([b1r, b1b])[None, :]
    g1 = jnp.concatenate([g1r, g1b])[None, :]
    bt1 = jnp.concatenate([bt1r, bt1b])[None, :]

    aidx3 = aidx.reshape(NB, 1, RB)
    psums, pcnt = _tc_pool(x, aidx.reshape(NBP, 1, RP))
    TE, Abf = _tc_tables(psums, pcnt, Wa, We, bA)
    Eg = _sc_gather(TE, eidx)
    H0 = _tc_h0a(x, rdf_feat, bdf_feat, aidx3,
                 Wx.astype(jnp.bfloat16), Wdr.astype(jnp.bfloat16),
                 Wdb.astype(jnp.bfloat16), Abf)
    st1 = _tc_stats1(H0, Eg)
    h2, st2 = _tc_layer2(H0, Eg, st1, g1, bt1,
                         W2.astype(jnp.bfloat16), b2[None, :])
    return _tc_norm2(h2, st2, g2[None, :], bt2[None, :])
